# baseline (speedup 1.0000x reference)
import functools

import jax
import jax.numpy as jnp
from jax import lax
from jax.experimental import pallas as pl
from jax.experimental.pallas import tpu as pltpu
from jax.experimental.pallas import tpu_sc as plsc

NUM_USERS = 100000
NUM_ITEMS = 1000000
D = 64
B = 16384

NC = 2
NS = 16
NW = NC * NS
BPW = B // NW
CHUNK = 128
RANGE = NUM_ITEMS // NW
OCAP = 1024
CW = 56
WIN = CW * 128
NWIN = 5
CTILES = NUM_ITEMS // 128
LASTWINSTART = (CTILES - CW) * 128
TAILSTART = CTILES * 128
TAILW = NUM_ITEMS - TAILSTART
BLK = 4096


def _body_user(uidx2d, utab2, up_out, uidx_v, keys_v, rows_v, sem):
    wid = lax.axis_index("s") * NC + lax.axis_index("c")
    pltpu.sync_copy(uidx2d.at[pl.ds(wid * (BPW // CHUNK), BPW // CHUNK)],
                    uidx_v)

    def keys(s, carry):
        c = s // 8
        g = s % 8
        u = uidx_v[c, pl.ds(g * 16, 16)]
        keys_v[c, pl.ds(g * 16, 16)] = jnp.right_shift(u, 1)
        return carry

    lax.fori_loop(0, (BPW // CHUNK) * 8, keys, 0)

    copies = []
    for c in range(BPW // CHUNK):
        copies.append(pltpu.async_copy(
            utab2.at[keys_v.at[c]],
            rows_v.at[pl.ds(c * CHUNK, CHUNK)], sem))
    for cp in copies:
        cp.wait()
    pltpu.sync_copy(rows_v, up_out.at[pl.ds(wid * BPW, BPW)])


def _body_item(uidx_hbm, iidx_hbm, itabT, up_hbm, part_out,
               win_v, tail_v, uch_v, u8_v, outb_v, acc_v,
               ii_v, uo_v, bo_v, blki_v, blku_v, ridx_v, shared_v, sem):
    core = lax.axis_index("c")
    sid = lax.axis_index("s")
    wid = sid * NC + core
    lo = wid * RANGE
    hi = lo + RANGE
    lo_c = lo // 128
    iota = lax.iota(jnp.int32, 16)
    zero = jnp.zeros((16,), jnp.float32)
    zeroi = jnp.zeros((16,), jnp.int32)

    def zout(g, carry):
        outb_v[g // 8, pl.ds((g % 8) * 16, 16)] = zero
        return carry

    lax.fori_loop(0, B // 16, zout, 0)

    def zridx(g, carry):
        ridx_v[pl.ds(g * 16, 16)] = g * 16 + iota
        return carry

    lax.fori_loop(0, (B // 128) // 16, zridx, 0)

    @pl.when(sid == 0)
    def _():
        pltpu.sync_copy(outb_v, shared_v)

    plsc.subcore_barrier()

    def scan_fill(rnd):
        rlo = rnd * OCAP

        def zlists(g, carry):
            ii_v[pl.ds(g * 16, 16)] = zeroi
            uo_v[pl.ds(g * 16, 16)] = zeroi
            bo_v[g // 8, pl.ds((g % 8) * 16, 16)] = zeroi
            return carry

        lax.fori_loop(0, OCAP // 16, zlists, 0)

        def blk_body(blk, tot):
            pltpu.sync_copy(iidx_hbm.at[pl.ds(blk * BLK, BLK)], blki_v)
            pltpu.sync_copy(uidx_hbm.at[pl.ds(blk * BLK, BLK)], blku_v)

            def step(s, tot):
                iv = blki_v[pl.ds(s * 16, 16)]
                uv = blku_v[pl.ds(s * 16, 16)]
                m = jnp.logical_and(iv >= lo, iv < hi)
                mi = m.astype(jnp.int32)
                rank = tot + plsc.cumsum(mi) - mi
                slot = rank - rlo
                keep = jnp.logical_and(
                    m, jnp.logical_and(slot >= 0, slot < OCAP))
                slot_c = jnp.clip(slot, 0, OCAP - 1)
                bvec = blk * BLK + s * 16 + iota
                plsc.store_scatter(ii_v, [slot_c], iv, mask=keep)
                plsc.store_scatter(uo_v, [slot_c], uv, mask=keep)
                plsc.store_scatter(
                    bo_v,
                    [jnp.right_shift(slot_c, 7),
                     jnp.bitwise_and(slot_c, 127)],
                    bvec, mask=keep)
                cnt = plsc.all_reduce_population_count(m)
                return tot + cnt

            return lax.fori_loop(0, BLK // 16, step, tot)

        tot = lax.fori_loop(0, B // BLK, blk_body, zeroi)
        return lax.reduce_max(tot, (0,))

    def process(total, rnd):
        cround = jnp.clip(total - rnd * OCAP, 0, OCAP)
        nch = (cround + CHUNK - 1) // CHUNK
        ngrp = nch * 8
        cround_v = jnp.full((16,), cround, jnp.int32)

        def zacc(g, carry):
            acc_v[pl.ds(g * 16, 16)] = zero
            return carry

        lax.fori_loop(0, OCAP // 16, zacc, 0)

        for r in range(8):
            def chb(ch, carry):
                pltpu.async_copy(up_hbm.at[bo_v.at[ch]], uch_v, sem).wait()

                def grp(g, carry2):
                    rows = g * 16 + iota
                    uo = uo_v[pl.ds(ch * CHUNK + g * 16, 16)]
                    uoff = jnp.left_shift(jnp.bitwise_and(uo, 1), 6)
                    for jj in range(8):
                        col = uoff + (r * 8 + jj)
                        u = plsc.load_gather(uch_v, [rows, col])
                        u8_v[jj, pl.ds(ch * CHUNK + g * 16, 16)] = (
                            jnp.maximum(u, 0.0))
                    return carry2

                lax.fori_loop(0, 8, grp, 0)
                return carry

            lax.fori_loop(0, nch, chb, 0)

            for w in range(NWIN):
                serve_lo = (lo_c + w * CW) * 128
                cstart = jnp.minimum(serve_lo, LASTWINSTART)
                cstart = pl.multiple_of(cstart, 128)
                pltpu.sync_copy(
                    itabT.at[pl.ds(r * 8, 8), pl.ds(cstart, WIN)], win_v)

                def grp2(g, carry):
                    e0 = g * 16
                    iv = ii_v[pl.ds(e0, 16)]
                    icol = iv - cstart
                    valid = jnp.logical_and(
                        jnp.logical_and(iv >= serve_lo, icol < WIN),
                        e0 + iota < cround_v)
                    icl = jnp.clip(icol, 0, WIN - 1)
                    a = zero
                    for jj in range(8):
                        jv = jnp.full((16,), jj, jnp.int32)
                        v = plsc.load_gather(win_v, [jv, icl])
                        u = u8_v[jj, pl.ds(e0, 16)]
                        a = a + jnp.where(
                            valid, jnp.maximum(v, 0.0) * u, 0.0)
                    acc_v[pl.ds(e0, 16)] = acc_v[pl.ds(e0, 16)] + a
                    return carry

                lax.fori_loop(0, ngrp, grp2, 0)

            pltpu.sync_copy(
                itabT.at[pl.ds(r * 8, 8), pl.ds(TAILSTART, TAILW)], tail_v)

            def grp3(g, carry):
                e0 = g * 16
                iv = ii_v[pl.ds(e0, 16)]
                icol = iv - TAILSTART
                valid = jnp.logical_and(
                    jnp.logical_and(icol >= 0, icol < TAILW),
                    e0 + iota < cround_v)
                icl = jnp.clip(icol, 0, TAILW - 1)
                a = zero
                for jj in range(8):
                    jv = jnp.full((16,), jj, jnp.int32)
                    v = plsc.load_gather(tail_v, [jv, icl])
                    u = u8_v[jj, pl.ds(e0, 16)]
                    a = a + jnp.where(valid, jnp.maximum(v, 0.0) * u, 0.0)
                acc_v[pl.ds(e0, 16)] = acc_v[pl.ds(e0, 16)] + a
                return carry

            lax.fori_loop(0, ngrp, grp3, 0)

        def sgrp(g, carry):
            e0 = g * 16
            bo = bo_v[g // 8, pl.ds((g % 8) * 16, 16)]
            ok = e0 + iota < cround_v
            plsc.store_scatter(
                outb_v,
                [jnp.right_shift(bo, 7), jnp.bitwise_and(bo, 127)],
                acc_v[pl.ds(e0, 16)], mask=ok)
            return carry

        lax.fori_loop(0, ngrp, sgrp, 0)

    def cond(state):
        rnd, total = state
        return rnd * OCAP < total

    def roundbody(state):
        rnd, _ = state
        total = scan_fill(rnd)
        process(total, rnd)
        return rnd + 1, total

    lax.while_loop(cond, roundbody, (0, 1))

    pltpu.sync_copy(outb_v, shared_v.at[ridx_v], add=True)
    plsc.subcore_barrier()
    pltpu.sync_copy(
        shared_v.at[pl.ds(sid * 8, 8)],
        part_out.at[core, pl.ds(sid * 8, 8)])


def _body_merge(part_hbm, out_hbm, a_v, b_v, f_v, sem):
    wid = lax.axis_index("s") * NC + lax.axis_index("c")
    rows = BPW // 128
    pltpu.sync_copy(part_hbm.at[0, pl.ds(wid * rows, rows)], a_v)
    pltpu.sync_copy(part_hbm.at[1, pl.ds(wid * rows, rows)], b_v)

    def grp(g, carry):
        r = g // 8
        cc = (g % 8) * 16
        f_v[pl.ds(g * 16, 16)] = (
            a_v[r, pl.ds(cc, 16)] + b_v[r, pl.ds(cc, 16)])
        return carry

    lax.fori_loop(0, BPW // 16, grp, 0)
    pltpu.sync_copy(f_v, out_hbm.at[pl.ds(wid * BPW, BPW)])


@functools.partial(jax.jit, static_argnums=())
def _run(uidx, iidx, user_table, item_table):
    mesh = plsc.VectorSubcoreMesh(core_axis_name="c", subcore_axis_name="s")
    cp = pltpu.CompilerParams(
        needs_layout_passes=False, use_tc_tiling_on_sc=True,
        disable_bounds_checks=True)

    uidx2d = uidx.reshape(B // CHUNK, CHUNK)
    utab2 = user_table.reshape(NUM_USERS // 2, 2 * D)
    itabT = item_table.T

    ku = pl.kernel(
        _body_user, mesh=mesh,
        out_type=jax.ShapeDtypeStruct((B, 2 * D), jnp.float32),
        scratch_types=[
            pltpu.VMEM((BPW // CHUNK, CHUNK), jnp.int32),
            pltpu.VMEM((BPW // CHUNK, CHUNK), jnp.int32),
            pltpu.VMEM((BPW, 2 * D), jnp.float32),
            pltpu.SemaphoreType.DMA,
        ],
        compiler_params=cp)
    up = ku(uidx2d, utab2)

    ki = pl.kernel(
        _body_item, mesh=mesh,
        out_type=jax.ShapeDtypeStruct((NC, B // 128, 128), jnp.float32),
        scratch_types=[
            pltpu.VMEM((8, WIN), jnp.float32),
            pltpu.VMEM((8, TAILW), jnp.float32),
            pltpu.VMEM((CHUNK, 2 * D), jnp.float32),
            pltpu.VMEM((8, OCAP), jnp.float32),
            pltpu.VMEM((B // 128, 128), jnp.float32),
            pltpu.VMEM((OCAP,), jnp.float32),
            pltpu.VMEM((OCAP,), jnp.int32),
            pltpu.VMEM((OCAP,), jnp.int32),
            pltpu.VMEM((OCAP // CHUNK, CHUNK), jnp.int32),
            pltpu.VMEM((BLK,), jnp.int32),
            pltpu.VMEM((BLK,), jnp.int32),
            pltpu.VMEM((B // 128,), jnp.int32),
            pltpu.VMEM_SHARED((B // 128, 128), jnp.float32),
            pltpu.SemaphoreType.DMA,
        ],
        compiler_params=cp)
    part = ki(uidx, iidx, itabT, up)

    km = pl.kernel(
        _body_merge, mesh=mesh,
        out_type=jax.ShapeDtypeStruct((B,), jnp.float32),
        scratch_types=[
            pltpu.VMEM((BPW // 128, 128), jnp.float32),
            pltpu.VMEM((BPW // 128, 128), jnp.float32),
            pltpu.VMEM((BPW,), jnp.float32),
            pltpu.SemaphoreType.DMA,
        ],
        compiler_params=cp)
    return km(part)


def kernel(user_indices, item_indices, user_table, item_table):
    return _run(user_indices.astype(jnp.int32),
                item_indices.astype(jnp.int32),
                user_table, item_table)


# speedup vs baseline: 2.6854x; 2.6854x over previous
import functools

import jax
import jax.numpy as jnp
from jax import lax
from jax.experimental import pallas as pl
from jax.experimental.pallas import tpu as pltpu
from jax.experimental.pallas import tpu_sc as plsc

NUM_USERS = 100000
NUM_ITEMS = 1000000
D = 64
B = 16384

NC = 2
NS = 16
NW = NC * NS
BPW = B // NW
CHUNK = 128
RANGE = NUM_ITEMS // NW
OCAP = 640
CW = 16
WIN = CW * 128
NWIN = 16
CTILES = NUM_ITEMS // 128
LASTWINSTART = (CTILES - CW) * 128
TAILSTART = CTILES * 128
TAILW = NUM_ITEMS - TAILSTART
BLK = 2048


def _body_user(uidx2d, utab2, up_out, uidx_v, keys_v, rows_v, sem):
    wid = lax.axis_index("s") * NC + lax.axis_index("c")
    pltpu.sync_copy(uidx2d.at[pl.ds(wid * (BPW // CHUNK), BPW // CHUNK)],
                    uidx_v)

    def keys(s, carry):
        c = s // 8
        g = s % 8
        u = uidx_v[c, pl.ds(g * 16, 16)]
        keys_v[c, pl.ds(g * 16, 16)] = jnp.right_shift(u, 1)
        return carry

    lax.fori_loop(0, (BPW // CHUNK) * 8, keys, 0)

    copies = []
    for c in range(BPW // CHUNK):
        copies.append(pltpu.async_copy(
            utab2.at[keys_v.at[c]],
            rows_v.at[pl.ds(c * CHUNK, CHUNK)], sem))
    for cp in copies:
        cp.wait()
    pltpu.sync_copy(rows_v, up_out.at[pl.ds(wid * BPW, BPW)])


def _body_item(uidx_hbm, iidx_hbm, itabT, up_hbm, part_out,
               win_v, tail_v, uch_v, uall_v, outb_v, acc_v,
               ii_v, uo_v, bo_v, ord_v, blki_v, blku_v, ridx_v,
               offs_s, shared_v, sem, semw):
    core = lax.axis_index("c")
    sid = lax.axis_index("s")
    wid = sid * NC + core
    lo = wid * RANGE
    hi = lo + RANGE
    lo_c = lo // 128
    iota = lax.iota(jnp.int32, 16)
    zero = jnp.zeros((16,), jnp.float32)
    zeroi = jnp.zeros((16,), jnp.int32)

    def zout(g, carry):
        outb_v[g // 8, pl.ds((g % 8) * 16, 16)] = zero
        return carry

    lax.fori_loop(0, B // 16, zout, 0)

    def zridx(g, carry):
        ridx_v[pl.ds(g * 16, 16)] = g * 16 + iota
        return carry

    lax.fori_loop(0, (B // 128) // 16, zridx, 0)

    @pl.when(sid == 0)
    def _():
        pltpu.sync_copy(outb_v, shared_v)

    plsc.subcore_barrier()

    def scan_fill(rnd):
        rlo = rnd * OCAP

        def zlists(g, carry):
            ii_v[pl.ds(g * 16, 16)] = zeroi
            uo_v[pl.ds(g * 16, 16)] = zeroi
            bo_v[g // 8, pl.ds((g % 8) * 16, 16)] = zeroi
            ord_v[pl.ds(g * 16, 16)] = zeroi
            return carry

        lax.fori_loop(0, OCAP // 16, zlists, 0)

        def blk_body(blk, tot):
            pltpu.sync_copy(iidx_hbm.at[pl.ds(blk * BLK, BLK)], blki_v)
            pltpu.sync_copy(uidx_hbm.at[pl.ds(blk * BLK, BLK)], blku_v)

            def step(s, tot):
                iv = blki_v[pl.ds(s * 16, 16)]
                uv = blku_v[pl.ds(s * 16, 16)]
                m = jnp.logical_and(iv >= lo, iv < hi)
                mi = m.astype(jnp.int32)
                rank = tot + plsc.cumsum(mi) - mi
                slot = rank - rlo
                keep = jnp.logical_and(
                    m, jnp.logical_and(slot >= 0, slot < OCAP))
                slot_c = jnp.clip(slot, 0, OCAP - 1)
                bvec = blk * BLK + s * 16 + iota
                plsc.store_scatter(ii_v, [slot_c], iv, mask=keep)
                plsc.store_scatter(uo_v, [slot_c], uv, mask=keep)
                plsc.store_scatter(
                    bo_v,
                    [jnp.right_shift(slot_c, 7),
                     jnp.bitwise_and(slot_c, 127)],
                    bvec, mask=keep)
                cnt = plsc.all_reduce_population_count(m)
                return tot + cnt

            return lax.fori_loop(0, BLK // 16, step, tot)

        tot = lax.fori_loop(0, B // BLK, blk_body, zeroi)
        return lax.reduce_max(tot, (0,))

    def process(total, rnd):
        cround = jnp.clip(total - rnd * OCAP, 0, OCAP)
        nch = (cround + CHUNK - 1) // CHUNK
        ngrp = (cround + 15) // 16
        cround_v = jnp.full((16,), cround, jnp.int32)

        def zacc(g, carry):
            acc_v[pl.ds(g * 16, 16)] = zero
            return carry

        lax.fori_loop(0, OCAP // 16, zacc, 0)

        def chb(ch, carry):
            pltpu.async_copy(up_hbm.at[bo_v.at[ch]], uch_v, sem).wait()

            def grp(g, carry2):
                rows = g * 16 + iota
                uo = uo_v[pl.ds(ch * CHUNK + g * 16, 16)]
                uoff = jnp.left_shift(jnp.bitwise_and(uo, 1), 6)
                for d in range(D):
                    u = plsc.load_gather(uch_v, [rows, uoff + d])
                    plsc.store_scatter(
                        uall_v, [jnp.full((16,), d, jnp.int32),
                                 ch * CHUNK + rows],
                        jnp.maximum(u, 0.0))
                return carry2

            lax.fori_loop(0, 8, grp, 0)
            return carry

        lax.fori_loop(0, nch, chb, 0)

        def bktw(w, off_splat):
            offs_s[w] = lax.reduce_max(off_splat, (0,))
            wfull = jnp.full((16,), w, jnp.int32)

            def bkt(g, tot):
                slots = g * 16 + iota
                iv = ii_v[pl.ds(g * 16, 16)]
                wb = jnp.right_shift(jnp.right_shift(iv, 7) - lo_c, 4)
                m = jnp.logical_and(
                    wb == wfull, g * 16 + iota < cround_v)
                mi = m.astype(jnp.int32)
                rank = tot + plsc.cumsum(mi) - mi
                plsc.store_scatter(
                    ord_v, [jnp.clip(rank, 0, OCAP - 1)], slots, mask=m)
                return tot + plsc.all_reduce_population_count(m)

            return lax.fori_loop(0, ngrp, bkt, off_splat)

        off_splat = lax.fori_loop(0, NWIN, bktw, zeroi)
        offs_s[NWIN] = lax.reduce_max(off_splat, (0,))

        def win_params(rw):
            w = lax.rem(rw, NWIN)
            r8 = (rw // NWIN) * 8
            serve = (lo_c + w * CW) * 128
            cst = pl.multiple_of(jnp.minimum(serve, LASTWINSTART), 128)
            return w, r8, serve, cst

        def start_win(rw):
            w, r8, serve, cst = win_params(rw)
            pltpu.async_copy(
                itabT.at[pl.ds(r8, 8), pl.ds(cst, WIN)],
                win_v.at[lax.rem(rw, 2)], semw)

        start_win(0)

        def rw_body(rw, carry):
            @pl.when(rw + 1 < 8 * NWIN)
            def _():
                start_win(rw + 1)

            pltpu.make_async_copy(
                itabT.at[pl.ds(0, 8), pl.ds(0, WIN)],
                win_v.at[0], semw).wait()

            w, r8, serve, cst = win_params(rw)
            buf = lax.rem(rw, 2)
            o_lo = offs_s[w]
            o_hi = offs_s[w + 1]
            glo = o_lo // 16
            ghi = (o_hi + 15) // 16
            olo = jnp.full((16,), o_lo, jnp.int32)
            ohi = jnp.full((16,), o_hi, jnp.int32)
            bufv = jnp.full((16,), buf, jnp.int32)
            servev = jnp.full((16,), serve, jnp.int32)
            cstv = jnp.full((16,), cst, jnp.int32)
            r8v = jnp.full((16,), r8, jnp.int32)

            def grp2(g, carry2):
                pos = g * 16 + iota
                slots = ord_v[pl.ds(g * 16, 16)]
                iv = plsc.load_gather(ii_v, [slots])
                icol = iv - cstv
                valid = jnp.logical_and(
                    jnp.logical_and(pos >= olo, pos < ohi),
                    jnp.logical_and(iv >= servev, icol < WIN))
                icl = jnp.clip(icol, 0, WIN - 1)
                a = zero
                for jj in range(8):
                    jv = jnp.full((16,), jj, jnp.int32)
                    v = plsc.load_gather(win_v, [bufv, jv, icl])
                    u = plsc.load_gather(uall_v, [r8v + jv, slots])
                    a = a + jnp.where(
                        valid, jnp.maximum(v, 0.0) * u, 0.0)
                prev = plsc.load_gather(acc_v, [slots])
                plsc.store_scatter(acc_v, [slots], prev + a, mask=valid)
                return carry2

            lax.fori_loop(glo, ghi, grp2, 0)
            return carry

        lax.fori_loop(0, 8 * NWIN, rw_body, 0)

        @pl.when(wid == NW - 1)
        def _():
            def tailr(r, carry):
                pltpu.sync_copy(
                    itabT.at[pl.ds(r * 8, 8), pl.ds(TAILSTART, TAILW)],
                    tail_v)
                r8v = jnp.full((16,), r * 8, jnp.int32)

                def grp3(g, carry2):
                    e0 = g * 16
                    iv = ii_v[pl.ds(e0, 16)]
                    icol = iv - TAILSTART
                    valid = jnp.logical_and(
                        jnp.logical_and(icol >= 0, icol < TAILW),
                        e0 + iota < cround_v)
                    icl = jnp.clip(icol, 0, TAILW - 1)
                    a = zero
                    for jj in range(8):
                        jv = jnp.full((16,), jj, jnp.int32)
                        v = plsc.load_gather(tail_v, [jv, icl])
                        u = plsc.load_gather(uall_v, [r8v + jv, e0 + iota])
                        a = a + jnp.where(
                            valid, jnp.maximum(v, 0.0) * u, 0.0)
                    acc_v[pl.ds(e0, 16)] = acc_v[pl.ds(e0, 16)] + a
                    return carry2

                lax.fori_loop(0, ngrp, grp3, 0)
                return carry

            lax.fori_loop(0, 8, tailr, 0)

        def sgrp(g, carry):
            e0 = g * 16
            bo = bo_v[g // 8, pl.ds((g % 8) * 16, 16)]
            ok = e0 + iota < cround_v
            plsc.store_scatter(
                outb_v,
                [jnp.right_shift(bo, 7), jnp.bitwise_and(bo, 127)],
                acc_v[pl.ds(e0, 16)], mask=ok)
            return carry

        lax.fori_loop(0, ngrp, sgrp, 0)

    def cond(state):
        rnd, total = state
        return rnd * OCAP < total

    def roundbody(state):
        rnd, _ = state
        total = scan_fill(rnd)
        process(total, rnd)
        return rnd + 1, total

    lax.while_loop(cond, roundbody, (0, 1))

    pltpu.sync_copy(outb_v, shared_v.at[ridx_v], add=True)
    plsc.subcore_barrier()
    pltpu.sync_copy(
        shared_v.at[pl.ds(sid * 8, 8)],
        part_out.at[core, pl.ds(sid * 8, 8)])


def _body_merge(part_hbm, out_hbm, a_v, b_v, f_v, sem):
    wid = lax.axis_index("s") * NC + lax.axis_index("c")
    rows = BPW // 128
    pltpu.sync_copy(part_hbm.at[0, pl.ds(wid * rows, rows)], a_v)
    pltpu.sync_copy(part_hbm.at[1, pl.ds(wid * rows, rows)], b_v)

    def grp(g, carry):
        r = g // 8
        cc = (g % 8) * 16
        f_v[pl.ds(g * 16, 16)] = (
            a_v[r, pl.ds(cc, 16)] + b_v[r, pl.ds(cc, 16)])
        return carry

    lax.fori_loop(0, BPW // 16, grp, 0)
    pltpu.sync_copy(f_v, out_hbm.at[pl.ds(wid * BPW, BPW)])


@functools.partial(jax.jit, static_argnums=())
def _run(uidx, iidx, user_table, item_table):
    mesh = plsc.VectorSubcoreMesh(core_axis_name="c", subcore_axis_name="s")
    cp = pltpu.CompilerParams(
        needs_layout_passes=False, use_tc_tiling_on_sc=True,
        disable_bounds_checks=True)

    uidx2d = uidx.reshape(B // CHUNK, CHUNK)
    utab2 = user_table.reshape(NUM_USERS // 2, 2 * D)
    itabT = item_table.T

    ku = pl.kernel(
        _body_user, mesh=mesh,
        out_type=jax.ShapeDtypeStruct((B, 2 * D), jnp.float32),
        scratch_types=[
            pltpu.VMEM((BPW // CHUNK, CHUNK), jnp.int32),
            pltpu.VMEM((BPW // CHUNK, CHUNK), jnp.int32),
            pltpu.VMEM((BPW, 2 * D), jnp.float32),
            pltpu.SemaphoreType.DMA,
        ],
        compiler_params=cp)
    up = ku(uidx2d, utab2)

    ki = pl.kernel(
        _body_item, mesh=mesh,
        out_type=jax.ShapeDtypeStruct((NC, B // 128, 128), jnp.float32),
        scratch_types=[
            pltpu.VMEM((2, 8, WIN), jnp.float32),
            pltpu.VMEM((8, TAILW), jnp.float32),
            pltpu.VMEM((CHUNK, 2 * D), jnp.float32),
            pltpu.VMEM((D, OCAP), jnp.float32),
            pltpu.VMEM((B // 128, 128), jnp.float32),
            pltpu.VMEM((OCAP,), jnp.float32),
            pltpu.VMEM((OCAP,), jnp.int32),
            pltpu.VMEM((OCAP,), jnp.int32),
            pltpu.VMEM((OCAP // CHUNK, CHUNK), jnp.int32),
            pltpu.VMEM((OCAP,), jnp.int32),
            pltpu.VMEM((BLK,), jnp.int32),
            pltpu.VMEM((BLK,), jnp.int32),
            pltpu.VMEM((B // 128,), jnp.int32),
            pltpu.SMEM((NWIN + 1,), jnp.int32),
            pltpu.VMEM_SHARED((B // 128, 128), jnp.float32),
            pltpu.SemaphoreType.DMA,
            pltpu.SemaphoreType.DMA,
        ],
        compiler_params=cp)
    part = ki(uidx, iidx, itabT, up)

    km = pl.kernel(
        _body_merge, mesh=mesh,
        out_type=jax.ShapeDtypeStruct((B,), jnp.float32),
        scratch_types=[
            pltpu.VMEM((BPW // 128, 128), jnp.float32),
            pltpu.VMEM((BPW // 128, 128), jnp.float32),
            pltpu.VMEM((BPW,), jnp.float32),
            pltpu.SemaphoreType.DMA,
        ],
        compiler_params=cp)
    return km(part)


def kernel(user_indices, item_indices, user_table, item_table):
    return _run(user_indices.astype(jnp.int32),
                item_indices.astype(jnp.int32),
                user_table, item_table)


# speedup vs baseline: 2.6999x; 1.0054x over previous
import functools

import jax
import jax.numpy as jnp
from jax import lax
from jax.experimental import pallas as pl
from jax.experimental.pallas import tpu as pltpu
from jax.experimental.pallas import tpu_sc as plsc

NUM_USERS = 100000
NUM_ITEMS = 1000000
D = 64
B = 16384

NC = 2
NS = 16
NW = NC * NS
BPW = B // NW
CHUNK = 128
RANGE = NUM_ITEMS // NW
OCAP = 640
CW = 24
WIN = CW * 128
NWIN = 11
CTILES = NUM_ITEMS // 128
LASTWINSTART = (CTILES - CW) * 128
TAILSTART = CTILES * 128
TAILW = NUM_ITEMS - TAILSTART
BLK = 1024


def _body_user(uidx2d, utab2, up_out, uidx_v, keys_v, rows_v, sem):
    wid = lax.axis_index("s") * NC + lax.axis_index("c")
    pltpu.sync_copy(uidx2d.at[pl.ds(wid * (BPW // CHUNK), BPW // CHUNK)],
                    uidx_v)

    def keys(s, carry):
        c = s // 8
        g = s % 8
        u = uidx_v[c, pl.ds(g * 16, 16)]
        keys_v[c, pl.ds(g * 16, 16)] = jnp.right_shift(u, 1)
        return carry

    lax.fori_loop(0, (BPW // CHUNK) * 8, keys, 0)

    copies = []
    for c in range(BPW // CHUNK):
        copies.append(pltpu.async_copy(
            utab2.at[keys_v.at[c]],
            rows_v.at[pl.ds(c * CHUNK, CHUNK)], sem))
    for cp in copies:
        cp.wait()
    pltpu.sync_copy(rows_v, up_out.at[pl.ds(wid * BPW, BPW)])


def _body_item(uidx_hbm, iidx_hbm, itabT, up_hbm, part_out,
               win_v, tail_v, uch_v, uall_v, outb_v, acc_v,
               ii_v, uo_v, bo_v, ord_v, blki_v, blku_v, ridx_v,
               offs_s, shared_v, sem, semw):
    core = lax.axis_index("c")
    sid = lax.axis_index("s")
    wid = sid * NC + core
    lo = wid * RANGE
    hi = lo + RANGE
    lo_c = lo // 128
    iota = lax.iota(jnp.int32, 16)
    zero = jnp.zeros((16,), jnp.float32)
    zeroi = jnp.zeros((16,), jnp.int32)

    def zout(g, carry):
        outb_v[g // 8, pl.ds((g % 8) * 16, 16)] = zero
        return carry

    lax.fori_loop(0, B // 16, zout, 0)

    def zridx(g, carry):
        ridx_v[pl.ds(g * 16, 16)] = g * 16 + iota
        return carry

    lax.fori_loop(0, (B // 128) // 16, zridx, 0)

    @pl.when(sid == 0)
    def _():
        pltpu.sync_copy(outb_v, shared_v)

    plsc.subcore_barrier()

    def scan_fill(rnd):
        rlo = rnd * OCAP

        def zlists(g, carry):
            ii_v[pl.ds(g * 16, 16)] = zeroi
            uo_v[pl.ds(g * 16, 16)] = zeroi
            bo_v[g // 8, pl.ds((g % 8) * 16, 16)] = zeroi
            ord_v[pl.ds(g * 16, 16)] = zeroi
            return carry

        lax.fori_loop(0, OCAP // 16, zlists, 0)

        def blk_body(blk, tot):
            pltpu.sync_copy(iidx_hbm.at[pl.ds(blk * BLK, BLK)], blki_v)
            pltpu.sync_copy(uidx_hbm.at[pl.ds(blk * BLK, BLK)], blku_v)

            def step(s, tot):
                iv = blki_v[pl.ds(s * 16, 16)]
                uv = blku_v[pl.ds(s * 16, 16)]
                m = jnp.logical_and(iv >= lo, iv < hi)
                mi = m.astype(jnp.int32)
                rank = tot + plsc.cumsum(mi) - mi
                slot = rank - rlo
                keep = jnp.logical_and(
                    m, jnp.logical_and(slot >= 0, slot < OCAP))
                slot_c = jnp.clip(slot, 0, OCAP - 1)
                bvec = blk * BLK + s * 16 + iota
                plsc.store_scatter(ii_v, [slot_c], iv, mask=keep)
                plsc.store_scatter(uo_v, [slot_c], uv, mask=keep)
                plsc.store_scatter(
                    bo_v,
                    [jnp.right_shift(slot_c, 7),
                     jnp.bitwise_and(slot_c, 127)],
                    bvec, mask=keep)
                cnt = plsc.all_reduce_population_count(m)
                return tot + cnt

            return lax.fori_loop(0, BLK // 16, step, tot)

        tot = lax.fori_loop(0, B // BLK, blk_body, zeroi)
        return lax.reduce_max(tot, (0,))

    def process(total, rnd):
        cround = jnp.clip(total - rnd * OCAP, 0, OCAP)
        nch = (cround + CHUNK - 1) // CHUNK
        ngrp = (cround + 15) // 16
        cround_v = jnp.full((16,), cround, jnp.int32)

        def zacc(g, carry):
            acc_v[pl.ds(g * 16, 16)] = zero
            return carry

        lax.fori_loop(0, OCAP // 16, zacc, 0)

        def chb(ch, carry):
            pltpu.async_copy(up_hbm.at[bo_v.at[ch]], uch_v, sem).wait()

            def grp(g, carry2):
                rows = g * 16 + iota
                uo = uo_v[pl.ds(ch * CHUNK + g * 16, 16)]
                uoff = jnp.left_shift(jnp.bitwise_and(uo, 1), 6)
                for d in range(D):
                    u = plsc.load_gather(uch_v, [rows, uoff + d])
                    plsc.store_scatter(
                        uall_v, [jnp.full((16,), d, jnp.int32),
                                 ch * CHUNK + rows],
                        jnp.maximum(u, 0.0))
                return carry2

            lax.fori_loop(0, 8, grp, 0)
            return carry

        lax.fori_loop(0, nch, chb, 0)

        def bktw(w, off_splat):
            offs_s[w] = lax.reduce_max(off_splat, (0,))
            clo = jnp.full((16,), w * CW, jnp.int32)
            chi = jnp.full((16,), (w + 1) * CW, jnp.int32)

            def bkt(g, tot):
                slots = g * 16 + iota
                iv = ii_v[pl.ds(g * 16, 16)]
                c_rel = jnp.right_shift(iv, 7) - lo_c
                m = jnp.logical_and(
                    jnp.logical_and(c_rel >= clo, c_rel < chi),
                    g * 16 + iota < cround_v)
                mi = m.astype(jnp.int32)
                rank = tot + plsc.cumsum(mi) - mi
                plsc.store_scatter(
                    ord_v, [jnp.clip(rank, 0, OCAP - 1)], slots, mask=m)
                return tot + plsc.all_reduce_population_count(m)

            return lax.fori_loop(0, ngrp, bkt, off_splat)

        off_splat = lax.fori_loop(0, NWIN, bktw, zeroi)
        offs_s[NWIN] = lax.reduce_max(off_splat, (0,))

        def win_params(rw):
            w = lax.rem(rw, NWIN)
            r8 = (rw // NWIN) * 8
            serve = (lo_c + w * CW) * 128
            cst = pl.multiple_of(jnp.minimum(serve, LASTWINSTART), 128)
            return w, r8, serve, cst

        def start_win(rw):
            w, r8, serve, cst = win_params(rw)
            pltpu.async_copy(
                itabT.at[pl.ds(r8, 8), pl.ds(cst, WIN)],
                win_v.at[lax.rem(rw, 2)], semw)

        start_win(0)

        def rw_body(rw, carry):
            @pl.when(rw + 1 < 8 * NWIN)
            def _():
                start_win(rw + 1)

            pltpu.make_async_copy(
                itabT.at[pl.ds(0, 8), pl.ds(0, WIN)],
                win_v.at[0], semw).wait()

            w, r8, serve, cst = win_params(rw)
            buf = lax.rem(rw, 2)
            o_lo = offs_s[w]
            o_hi = offs_s[w + 1]
            glo = o_lo // 16
            ghi = (o_hi + 15) // 16
            olo = jnp.full((16,), o_lo, jnp.int32)
            ohi = jnp.full((16,), o_hi, jnp.int32)
            bufv = jnp.full((16,), buf, jnp.int32)
            servev = jnp.full((16,), serve, jnp.int32)
            cstv = jnp.full((16,), cst, jnp.int32)
            r8v = jnp.full((16,), r8, jnp.int32)

            def grp2(g, carry2):
                pos = g * 16 + iota
                slots = ord_v[pl.ds(g * 16, 16)]
                iv = plsc.load_gather(ii_v, [slots])
                icol = iv - cstv
                valid = jnp.logical_and(
                    jnp.logical_and(pos >= olo, pos < ohi),
                    jnp.logical_and(iv >= servev, icol < WIN))
                icl = jnp.clip(icol, 0, WIN - 1)
                a = zero
                for jj in range(8):
                    jv = jnp.full((16,), jj, jnp.int32)
                    v = plsc.load_gather(win_v, [bufv, jv, icl])
                    u = plsc.load_gather(uall_v, [r8v + jv, slots])
                    a = a + jnp.where(
                        valid, jnp.maximum(v, 0.0) * u, 0.0)
                prev = plsc.load_gather(acc_v, [slots])
                plsc.store_scatter(acc_v, [slots], prev + a, mask=valid)
                return carry2

            lax.fori_loop(glo, ghi, grp2, 0)
            return carry

        lax.fori_loop(0, 8 * NWIN, rw_body, 0)

        @pl.when(wid == NW - 1)
        def _():
            def tailr(r, carry):
                pltpu.sync_copy(
                    itabT.at[pl.ds(r * 8, 8), pl.ds(TAILSTART, TAILW)],
                    tail_v)
                r8v = jnp.full((16,), r * 8, jnp.int32)

                def grp3(g, carry2):
                    e0 = g * 16
                    iv = ii_v[pl.ds(e0, 16)]
                    icol = iv - TAILSTART
                    valid = jnp.logical_and(
                        jnp.logical_and(icol >= 0, icol < TAILW),
                        e0 + iota < cround_v)
                    icl = jnp.clip(icol, 0, TAILW - 1)
                    a = zero
                    for jj in range(8):
                        jv = jnp.full((16,), jj, jnp.int32)
                        v = plsc.load_gather(tail_v, [jv, icl])
                        u = plsc.load_gather(uall_v, [r8v + jv, e0 + iota])
                        a = a + jnp.where(
                            valid, jnp.maximum(v, 0.0) * u, 0.0)
                    acc_v[pl.ds(e0, 16)] = acc_v[pl.ds(e0, 16)] + a
                    return carry2

                lax.fori_loop(0, ngrp, grp3, 0)
                return carry

            lax.fori_loop(0, 8, tailr, 0)

        def sgrp(g, carry):
            e0 = g * 16
            bo = bo_v[g // 8, pl.ds((g % 8) * 16, 16)]
            ok = e0 + iota < cround_v
            plsc.store_scatter(
                outb_v,
                [jnp.right_shift(bo, 7), jnp.bitwise_and(bo, 127)],
                acc_v[pl.ds(e0, 16)], mask=ok)
            return carry

        lax.fori_loop(0, ngrp, sgrp, 0)

    def cond(state):
        rnd, total = state
        return rnd * OCAP < total

    def roundbody(state):
        rnd, _ = state
        total = scan_fill(rnd)
        process(total, rnd)
        return rnd + 1, total

    lax.while_loop(cond, roundbody, (0, 1))

    pltpu.sync_copy(outb_v, shared_v.at[ridx_v], add=True)
    plsc.subcore_barrier()
    pltpu.sync_copy(
        shared_v.at[pl.ds(sid * 8, 8)],
        part_out.at[core, pl.ds(sid * 8, 8)])


def _body_merge(part_hbm, out_hbm, a_v, b_v, f_v, sem):
    wid = lax.axis_index("s") * NC + lax.axis_index("c")
    rows = BPW // 128
    pltpu.sync_copy(part_hbm.at[0, pl.ds(wid * rows, rows)], a_v)
    pltpu.sync_copy(part_hbm.at[1, pl.ds(wid * rows, rows)], b_v)

    def grp(g, carry):
        r = g // 8
        cc = (g % 8) * 16
        f_v[pl.ds(g * 16, 16)] = (
            a_v[r, pl.ds(cc, 16)] + b_v[r, pl.ds(cc, 16)])
        return carry

    lax.fori_loop(0, BPW // 16, grp, 0)
    pltpu.sync_copy(f_v, out_hbm.at[pl.ds(wid * BPW, BPW)])


@functools.partial(jax.jit, static_argnums=())
def _run(uidx, iidx, user_table, item_table):
    mesh = plsc.VectorSubcoreMesh(core_axis_name="c", subcore_axis_name="s")
    cp = pltpu.CompilerParams(
        needs_layout_passes=False, use_tc_tiling_on_sc=True,
        disable_bounds_checks=True)

    uidx2d = uidx.reshape(B // CHUNK, CHUNK)
    utab2 = user_table.reshape(NUM_USERS // 2, 2 * D)
    itabT = item_table.T

    ku = pl.kernel(
        _body_user, mesh=mesh,
        out_type=jax.ShapeDtypeStruct((B, 2 * D), jnp.float32),
        scratch_types=[
            pltpu.VMEM((BPW // CHUNK, CHUNK), jnp.int32),
            pltpu.VMEM((BPW // CHUNK, CHUNK), jnp.int32),
            pltpu.VMEM((BPW, 2 * D), jnp.float32),
            pltpu.SemaphoreType.DMA,
        ],
        compiler_params=cp)
    up = ku(uidx2d, utab2)

    ki = pl.kernel(
        _body_item, mesh=mesh,
        out_type=jax.ShapeDtypeStruct((NC, B // 128, 128), jnp.float32),
        scratch_types=[
            pltpu.VMEM((2, 8, WIN), jnp.float32),
            pltpu.VMEM((8, TAILW), jnp.float32),
            pltpu.VMEM((CHUNK, 2 * D), jnp.float32),
            pltpu.VMEM((D, OCAP), jnp.float32),
            pltpu.VMEM((B // 128, 128), jnp.float32),
            pltpu.VMEM((OCAP,), jnp.float32),
            pltpu.VMEM((OCAP,), jnp.int32),
            pltpu.VMEM((OCAP,), jnp.int32),
            pltpu.VMEM((OCAP // CHUNK, CHUNK), jnp.int32),
            pltpu.VMEM((OCAP,), jnp.int32),
            pltpu.VMEM((BLK,), jnp.int32),
            pltpu.VMEM((BLK,), jnp.int32),
            pltpu.VMEM((B // 128,), jnp.int32),
            pltpu.SMEM((NWIN + 1,), jnp.int32),
            pltpu.VMEM_SHARED((B // 128, 128), jnp.float32),
            pltpu.SemaphoreType.DMA,
            pltpu.SemaphoreType.DMA,
        ],
        compiler_params=cp)
    part = ki(uidx, iidx, itabT, up)

    km = pl.kernel(
        _body_merge, mesh=mesh,
        out_type=jax.ShapeDtypeStruct((B,), jnp.float32),
        scratch_types=[
            pltpu.VMEM((BPW // 128, 128), jnp.float32),
            pltpu.VMEM((BPW // 128, 128), jnp.float32),
            pltpu.VMEM((BPW,), jnp.float32),
            pltpu.SemaphoreType.DMA,
        ],
        compiler_params=cp)
    return km(part)


def kernel(user_indices, item_indices, user_table, item_table):
    return _run(user_indices.astype(jnp.int32),
                item_indices.astype(jnp.int32),
                user_table, item_table)


# speedup vs baseline: 2.7610x; 1.0226x over previous
import functools

import jax
import jax.numpy as jnp
from jax import lax
from jax.experimental import pallas as pl
from jax.experimental.pallas import tpu as pltpu
from jax.experimental.pallas import tpu_sc as plsc

NUM_USERS = 100000
NUM_ITEMS = 1000000
D = 64
B = 16384

NC = 2
NS = 16
NW = NC * NS
BPW = B // NW
CHUNK = 128
RANGE = NUM_ITEMS // NW
OCAP = 640
CW = 24
WIN = CW * 128
NWIN = 11
CTILES = NUM_ITEMS // 128
LASTWINSTART = (CTILES - CW) * 128
TAILSTART = CTILES * 128
TAILW = NUM_ITEMS - TAILSTART
BLK = 1024


def _body_user(uidx2d, utab2, up_out, uidx_v, keys_v, rows_v, sem):
    wid = lax.axis_index("s") * NC + lax.axis_index("c")
    pltpu.sync_copy(uidx2d.at[pl.ds(wid * (BPW // CHUNK), BPW // CHUNK)],
                    uidx_v)

    def keys(s, carry):
        c = s // 8
        g = s % 8
        u = uidx_v[c, pl.ds(g * 16, 16)]
        keys_v[c, pl.ds(g * 16, 16)] = jnp.right_shift(u, 1)
        return carry

    lax.fori_loop(0, (BPW // CHUNK) * 8, keys, 0)

    copies = []
    for c in range(BPW // CHUNK):
        copies.append(pltpu.async_copy(
            utab2.at[keys_v.at[c]],
            rows_v.at[pl.ds(c * CHUNK, CHUNK)], sem))
    for cp in copies:
        cp.wait()
    pltpu.sync_copy(rows_v, up_out.at[pl.ds(wid * BPW, BPW)])


def _body_item(uidx_hbm, iidx_hbm, itabT, up_hbm, part_out,
               win_v, tail_v, uch_v, uall_v, outb_v, acc_v,
               ii_v, uo_v, bo_v, ord_v, blki_v, blku_v, ridx_v,
               offs_s, shared_v, sem, semw):
    core = lax.axis_index("c")
    sid = lax.axis_index("s")
    wid = sid * NC + core
    lo = wid * RANGE
    hi = lo + RANGE
    lo_c = lo // 128
    iota = lax.iota(jnp.int32, 16)
    zero = jnp.zeros((16,), jnp.float32)
    zeroi = jnp.zeros((16,), jnp.int32)

    def zout(g, carry):
        outb_v[g // 8, pl.ds((g % 8) * 16, 16)] = zero
        return carry

    lax.fori_loop(0, B // 16, zout, 0)

    def zridx(g, carry):
        ridx_v[pl.ds(g * 16, 16)] = g * 16 + iota
        return carry

    lax.fori_loop(0, (B // 128) // 16, zridx, 0)

    @pl.when(sid == 0)
    def _():
        pltpu.sync_copy(outb_v, shared_v)

    plsc.subcore_barrier()

    def win_params(rw):
        w = lax.rem(rw, NWIN)
        r8 = (rw // NWIN) * 8
        serve = (lo_c + w * CW) * 128
        cst = pl.multiple_of(jnp.minimum(serve, LASTWINSTART), 128)
        return w, r8, serve, cst

    def start_win(rw):
        w, r8, serve, cst = win_params(rw)
        pltpu.async_copy(
            itabT.at[pl.ds(r8, 8), pl.ds(cst, WIN)],
            win_v.at[lax.rem(rw, 2)], semw)

    def scan_fill(rnd):
        rlo = rnd * OCAP

        def zlists(g, carry):
            ii_v[pl.ds(g * 16, 16)] = zeroi
            uo_v[pl.ds(g * 16, 16)] = zeroi
            bo_v[g // 8, pl.ds((g % 8) * 16, 16)] = zeroi
            ord_v[pl.ds(g * 16, 16)] = zeroi
            return carry

        lax.fori_loop(0, OCAP // 16, zlists, 0)

        def blk_body(blk, tot):
            c1 = pltpu.async_copy(
                iidx_hbm.at[pl.ds(blk * BLK, BLK)], blki_v, sem)
            c2 = pltpu.async_copy(
                uidx_hbm.at[pl.ds(blk * BLK, BLK)], blku_v, sem)
            c1.wait()
            c2.wait()

            def step(s, tot):
                iv = blki_v[pl.ds(s * 16, 16)]
                uv = blku_v[pl.ds(s * 16, 16)]
                m = jnp.logical_and(iv >= lo, iv < hi)
                mi = m.astype(jnp.int32)
                rank = tot + plsc.cumsum(mi) - mi
                slot = rank - rlo
                keep = jnp.logical_and(
                    m, jnp.logical_and(slot >= 0, slot < OCAP))
                slot_c = jnp.clip(slot, 0, OCAP - 1)
                bvec = blk * BLK + s * 16 + iota
                plsc.store_scatter(ii_v, [slot_c], iv, mask=keep)
                plsc.store_scatter(uo_v, [slot_c], uv, mask=keep)
                plsc.store_scatter(
                    bo_v,
                    [jnp.right_shift(slot_c, 7),
                     jnp.bitwise_and(slot_c, 127)],
                    bvec, mask=keep)
                cnt = plsc.all_reduce_population_count(m)
                return tot + cnt

            return lax.fori_loop(0, BLK // 16, step, tot)

        tot = lax.fori_loop(0, B // BLK, blk_body, zeroi)
        return lax.reduce_max(tot, (0,))

    def process(total, rnd):
        cround = jnp.clip(total - rnd * OCAP, 0, OCAP)
        nch = (cround + CHUNK - 1) // CHUNK
        ngrp = (cround + 15) // 16
        cround_v = jnp.full((16,), cround, jnp.int32)

        def zacc(g, carry):
            acc_v[pl.ds(g * 16, 16)] = zero
            return carry

        lax.fori_loop(0, OCAP // 16, zacc, 0)

        def chb(ch, carry):
            pltpu.async_copy(up_hbm.at[bo_v.at[ch]], uch_v, sem).wait()

            def grp(g, carry2):
                rows = g * 16 + iota
                uo = uo_v[pl.ds(ch * CHUNK + g * 16, 16)]
                uoff = jnp.left_shift(jnp.bitwise_and(uo, 1), 6)
                for d in range(D):
                    u = plsc.load_gather(uch_v, [rows, uoff + d])
                    plsc.store_scatter(
                        uall_v, [jnp.full((16,), d, jnp.int32),
                                 ch * CHUNK + rows],
                        jnp.maximum(u, 0.0))
                return carry2

            lax.fori_loop(0, 8, grp, 0)
            return carry

        lax.fori_loop(0, nch, chb, 0)

        def bktw(w, off_splat):
            offs_s[w] = lax.reduce_max(off_splat, (0,))
            clo = jnp.full((16,), w * CW, jnp.int32)
            chi = jnp.full((16,), (w + 1) * CW, jnp.int32)

            def bkt(g, tot):
                slots = g * 16 + iota
                iv = ii_v[pl.ds(g * 16, 16)]
                c_rel = jnp.right_shift(iv, 7) - lo_c
                m = jnp.logical_and(
                    jnp.logical_and(c_rel >= clo, c_rel < chi),
                    g * 16 + iota < cround_v)
                mi = m.astype(jnp.int32)
                rank = tot + plsc.cumsum(mi) - mi
                plsc.store_scatter(
                    ord_v, [jnp.clip(rank, 0, OCAP - 1)], slots, mask=m)
                return tot + plsc.all_reduce_population_count(m)

            return lax.fori_loop(0, ngrp, bkt, off_splat)

        off_splat = lax.fori_loop(0, NWIN, bktw, zeroi)
        offs_s[NWIN] = lax.reduce_max(off_splat, (0,))

        def rw_body(rw, carry):
            pltpu.make_async_copy(
                itabT.at[pl.ds(0, 8), pl.ds(0, WIN)],
                win_v.at[0], semw).wait()

            w, r8, serve, cst = win_params(rw)
            buf = lax.rem(rw, 2)
            o_lo = offs_s[w]
            o_hi = offs_s[w + 1]
            glo = o_lo // 16
            ghi = (o_hi + 15) // 16
            olo = jnp.full((16,), o_lo, jnp.int32)
            ohi = jnp.full((16,), o_hi, jnp.int32)
            bufv = jnp.full((16,), buf, jnp.int32)
            servev = jnp.full((16,), serve, jnp.int32)
            cstv = jnp.full((16,), cst, jnp.int32)
            r8v = jnp.full((16,), r8, jnp.int32)

            def grp2(g, carry2):
                pos = g * 16 + iota
                slots = ord_v[pl.ds(g * 16, 16)]
                iv = plsc.load_gather(ii_v, [slots])
                icol = iv - cstv
                valid = jnp.logical_and(
                    jnp.logical_and(pos >= olo, pos < ohi),
                    jnp.logical_and(iv >= servev, icol < WIN))
                icl = jnp.clip(icol, 0, WIN - 1)
                a = zero
                for jj in range(8):
                    jv = jnp.full((16,), jj, jnp.int32)
                    v = plsc.load_gather(win_v, [bufv, jv, icl])
                    u = plsc.load_gather(uall_v, [r8v + jv, slots])
                    a = a + jnp.where(
                        valid, jnp.maximum(v, 0.0) * u, 0.0)
                prev = plsc.load_gather(acc_v, [slots])
                plsc.store_scatter(acc_v, [slots], prev + a, mask=valid)
                return carry2

            lax.fori_loop(glo, ghi, grp2, 0)

            @pl.when(rw + 2 < 8 * NWIN)
            def _():
                start_win(rw + 2)

            return carry

        lax.fori_loop(0, 8 * NWIN, rw_body, 0)

        @pl.when(wid == NW - 1)
        def _():
            def tailr(r, carry):
                pltpu.sync_copy(
                    itabT.at[pl.ds(r * 8, 8), pl.ds(TAILSTART, TAILW)],
                    tail_v)
                r8v = jnp.full((16,), r * 8, jnp.int32)

                def grp3(g, carry2):
                    e0 = g * 16
                    iv = ii_v[pl.ds(e0, 16)]
                    icol = iv - TAILSTART
                    valid = jnp.logical_and(
                        jnp.logical_and(icol >= 0, icol < TAILW),
                        e0 + iota < cround_v)
                    icl = jnp.clip(icol, 0, TAILW - 1)
                    a = zero
                    for jj in range(8):
                        jv = jnp.full((16,), jj, jnp.int32)
                        v = plsc.load_gather(tail_v, [jv, icl])
                        u = plsc.load_gather(uall_v, [r8v + jv, e0 + iota])
                        a = a + jnp.where(
                            valid, jnp.maximum(v, 0.0) * u, 0.0)
                    acc_v[pl.ds(e0, 16)] = acc_v[pl.ds(e0, 16)] + a
                    return carry2

                lax.fori_loop(0, ngrp, grp3, 0)
                return carry

            lax.fori_loop(0, 8, tailr, 0)

        def sgrp(g, carry):
            e0 = g * 16
            bo = bo_v[g // 8, pl.ds((g % 8) * 16, 16)]
            ok = e0 + iota < cround_v
            plsc.store_scatter(
                outb_v,
                [jnp.right_shift(bo, 7), jnp.bitwise_and(bo, 127)],
                acc_v[pl.ds(e0, 16)], mask=ok)
            return carry

        lax.fori_loop(0, ngrp, sgrp, 0)

    def cond(state):
        rnd, total = state
        return rnd * OCAP < total

    def roundbody(state):
        rnd, _ = state
        start_win(0)
        start_win(1)
        total = scan_fill(rnd)
        process(total, rnd)
        return rnd + 1, total

    lax.while_loop(cond, roundbody, (0, 1))

    pltpu.sync_copy(outb_v, shared_v.at[ridx_v], add=True)
    plsc.subcore_barrier()
    pltpu.sync_copy(
        shared_v.at[pl.ds(sid * 8, 8)],
        part_out.at[core, pl.ds(sid * 8, 8)])


def _body_merge(part_hbm, out_hbm, a_v, b_v, f_v, sem):
    wid = lax.axis_index("s") * NC + lax.axis_index("c")
    rows = BPW // 128
    pltpu.sync_copy(part_hbm.at[0, pl.ds(wid * rows, rows)], a_v)
    pltpu.sync_copy(part_hbm.at[1, pl.ds(wid * rows, rows)], b_v)

    def grp(g, carry):
        r = g // 8
        cc = (g % 8) * 16
        f_v[pl.ds(g * 16, 16)] = (
            a_v[r, pl.ds(cc, 16)] + b_v[r, pl.ds(cc, 16)])
        return carry

    lax.fori_loop(0, BPW // 16, grp, 0)
    pltpu.sync_copy(f_v, out_hbm.at[pl.ds(wid * BPW, BPW)])


@functools.partial(jax.jit, static_argnums=())
def _run(uidx, iidx, user_table, item_table):
    mesh = plsc.VectorSubcoreMesh(core_axis_name="c", subcore_axis_name="s")
    cp = pltpu.CompilerParams(
        needs_layout_passes=False, use_tc_tiling_on_sc=True,
        disable_bounds_checks=True)

    uidx2d = uidx.reshape(B // CHUNK, CHUNK)
    utab2 = user_table.reshape(NUM_USERS // 2, 2 * D)
    itabT = item_table.T

    ku = pl.kernel(
        _body_user, mesh=mesh,
        out_type=jax.ShapeDtypeStruct((B, 2 * D), jnp.float32),
        scratch_types=[
            pltpu.VMEM((BPW // CHUNK, CHUNK), jnp.int32),
            pltpu.VMEM((BPW // CHUNK, CHUNK), jnp.int32),
            pltpu.VMEM((BPW, 2 * D), jnp.float32),
            pltpu.SemaphoreType.DMA,
        ],
        compiler_params=cp)
    up = ku(uidx2d, utab2)

    ki = pl.kernel(
        _body_item, mesh=mesh,
        out_type=jax.ShapeDtypeStruct((NC, B // 128, 128), jnp.float32),
        scratch_types=[
            pltpu.VMEM((2, 8, WIN), jnp.float32),
            pltpu.VMEM((8, TAILW), jnp.float32),
            pltpu.VMEM((CHUNK, 2 * D), jnp.float32),
            pltpu.VMEM((D, OCAP), jnp.float32),
            pltpu.VMEM((B // 128, 128), jnp.float32),
            pltpu.VMEM((OCAP,), jnp.float32),
            pltpu.VMEM((OCAP,), jnp.int32),
            pltpu.VMEM((OCAP,), jnp.int32),
            pltpu.VMEM((OCAP // CHUNK, CHUNK), jnp.int32),
            pltpu.VMEM((OCAP,), jnp.int32),
            pltpu.VMEM((BLK,), jnp.int32),
            pltpu.VMEM((BLK,), jnp.int32),
            pltpu.VMEM((B // 128,), jnp.int32),
            pltpu.SMEM((NWIN + 1,), jnp.int32),
            pltpu.VMEM_SHARED((B // 128, 128), jnp.float32),
            pltpu.SemaphoreType.DMA,
            pltpu.SemaphoreType.DMA,
        ],
        compiler_params=cp)
    part = ki(uidx, iidx, itabT, up)

    km = pl.kernel(
        _body_merge, mesh=mesh,
        out_type=jax.ShapeDtypeStruct((B,), jnp.float32),
        scratch_types=[
            pltpu.VMEM((BPW // 128, 128), jnp.float32),
            pltpu.VMEM((BPW // 128, 128), jnp.float32),
            pltpu.VMEM((BPW,), jnp.float32),
            pltpu.SemaphoreType.DMA,
        ],
        compiler_params=cp)
    return km(part)


def kernel(user_indices, item_indices, user_table, item_table):
    return _run(user_indices.astype(jnp.int32),
                item_indices.astype(jnp.int32),
                user_table, item_table)


# speedup vs baseline: 2.7732x; 1.0044x over previous
import functools

import jax
import jax.numpy as jnp
from jax import lax
from jax.experimental import pallas as pl
from jax.experimental.pallas import tpu as pltpu
from jax.experimental.pallas import tpu_sc as plsc

NUM_USERS = 100000
NUM_ITEMS = 1000000
D = 64
B = 16384

NC = 2
NS = 16
NW = NC * NS
BPW = B // NW
CHUNK = 128
RANGE = NUM_ITEMS // NW
OCAP = 640
CW = 12
DEPTH = 4
WIN = CW * 128
NWIN = 21
CTILES = NUM_ITEMS // 128
LASTWINSTART = (CTILES - CW) * 128
TAILSTART = CTILES * 128
TAILW = NUM_ITEMS - TAILSTART
BLK = 1024


def _body_user(uidx2d, utab2, up_out, uidx_v, keys_v, rows_v, sem):
    wid = lax.axis_index("s") * NC + lax.axis_index("c")
    pltpu.sync_copy(uidx2d.at[pl.ds(wid * (BPW // CHUNK), BPW // CHUNK)],
                    uidx_v)

    def keys(s, carry):
        c = s // 8
        g = s % 8
        u = uidx_v[c, pl.ds(g * 16, 16)]
        keys_v[c, pl.ds(g * 16, 16)] = jnp.right_shift(u, 1)
        return carry

    lax.fori_loop(0, (BPW // CHUNK) * 8, keys, 0)

    copies = []
    for c in range(BPW // CHUNK):
        copies.append(pltpu.async_copy(
            utab2.at[keys_v.at[c]],
            rows_v.at[pl.ds(c * CHUNK, CHUNK)], sem))
    for cp in copies:
        cp.wait()
    pltpu.sync_copy(rows_v, up_out.at[pl.ds(wid * BPW, BPW)])


def _body_item(uidx_hbm, iidx_hbm, itabT, up_hbm, part_out,
               win_v, tail_v, uch_v, uall_v, outb_v, acc_v,
               ii_v, uo_v, bo_v, ord_v, blki_v, blku_v, ridx_v,
               offs_s, shared_v, sem, semw):
    core = lax.axis_index("c")
    sid = lax.axis_index("s")
    wid = sid * NC + core
    lo = wid * RANGE
    hi = lo + RANGE
    lo_c = lo // 128
    iota = lax.iota(jnp.int32, 16)
    zero = jnp.zeros((16,), jnp.float32)
    zeroi = jnp.zeros((16,), jnp.int32)

    def zout(g, carry):
        outb_v[g // 8, pl.ds((g % 8) * 16, 16)] = zero
        return carry

    lax.fori_loop(0, B // 16, zout, 0)

    def zridx(g, carry):
        ridx_v[pl.ds(g * 16, 16)] = g * 16 + iota
        return carry

    lax.fori_loop(0, (B // 128) // 16, zridx, 0)

    @pl.when(sid == 0)
    def _():
        pltpu.sync_copy(outb_v, shared_v)

    plsc.subcore_barrier()

    def win_params(rw):
        w = lax.rem(rw, NWIN)
        r8 = (rw // NWIN) * 8
        serve = (lo_c + w * CW) * 128
        cst = pl.multiple_of(jnp.minimum(serve, LASTWINSTART), 128)
        return w, r8, serve, cst

    def start_win(rw):
        w, r8, serve, cst = win_params(rw)
        pltpu.async_copy(
            itabT.at[pl.ds(r8, 8), pl.ds(cst, WIN)],
            win_v.at[lax.rem(rw, DEPTH)], semw)

    def scan_fill(rnd):
        rlo = rnd * OCAP

        def zlists(g, carry):
            ii_v[pl.ds(g * 16, 16)] = zeroi
            uo_v[pl.ds(g * 16, 16)] = zeroi
            bo_v[g // 8, pl.ds((g % 8) * 16, 16)] = zeroi
            ord_v[pl.ds(g * 16, 16)] = zeroi
            return carry

        lax.fori_loop(0, OCAP // 16, zlists, 0)

        def blk_body(blk, tot):
            c1 = pltpu.async_copy(
                iidx_hbm.at[pl.ds(blk * BLK, BLK)], blki_v, sem)
            c2 = pltpu.async_copy(
                uidx_hbm.at[pl.ds(blk * BLK, BLK)], blku_v, sem)
            c1.wait()
            c2.wait()

            def step(s, tot):
                iv = blki_v[pl.ds(s * 16, 16)]
                uv = blku_v[pl.ds(s * 16, 16)]
                m = jnp.logical_and(iv >= lo, iv < hi)
                mi = m.astype(jnp.int32)
                rank = tot + plsc.cumsum(mi) - mi
                slot = rank - rlo
                keep = jnp.logical_and(
                    m, jnp.logical_and(slot >= 0, slot < OCAP))
                slot_c = jnp.clip(slot, 0, OCAP - 1)
                bvec = blk * BLK + s * 16 + iota
                plsc.store_scatter(ii_v, [slot_c], iv, mask=keep)
                plsc.store_scatter(uo_v, [slot_c], uv, mask=keep)
                plsc.store_scatter(
                    bo_v,
                    [jnp.right_shift(slot_c, 7),
                     jnp.bitwise_and(slot_c, 127)],
                    bvec, mask=keep)
                cnt = plsc.all_reduce_population_count(m)
                return tot + cnt

            return lax.fori_loop(0, BLK // 16, step, tot)

        tot = lax.fori_loop(0, B // BLK, blk_body, zeroi)
        return lax.reduce_max(tot, (0,))

    def process(total, rnd):
        cround = jnp.clip(total - rnd * OCAP, 0, OCAP)
        nch = (cround + CHUNK - 1) // CHUNK
        ngrp = (cround + 15) // 16
        cround_v = jnp.full((16,), cround, jnp.int32)

        def zacc(g, carry):
            acc_v[pl.ds(g * 16, 16)] = zero
            return carry

        lax.fori_loop(0, OCAP // 16, zacc, 0)

        def chb(ch, carry):
            pltpu.async_copy(up_hbm.at[bo_v.at[ch]], uch_v, sem).wait()

            def grp(g, carry2):
                rows = g * 16 + iota
                uo = uo_v[pl.ds(ch * CHUNK + g * 16, 16)]
                uoff = jnp.left_shift(jnp.bitwise_and(uo, 1), 6)
                for d in range(D):
                    u = plsc.load_gather(uch_v, [rows, uoff + d])
                    plsc.store_scatter(
                        uall_v, [jnp.full((16,), d, jnp.int32),
                                 ch * CHUNK + rows],
                        jnp.maximum(u, 0.0))
                return carry2

            lax.fori_loop(0, 8, grp, 0)
            return carry

        lax.fori_loop(0, nch, chb, 0)

        def bktw(w, off_splat):
            offs_s[w] = lax.reduce_max(off_splat, (0,))
            clo = jnp.full((16,), w * CW, jnp.int32)
            chi = jnp.full((16,), (w + 1) * CW, jnp.int32)

            def bkt(g, tot):
                slots = g * 16 + iota
                iv = ii_v[pl.ds(g * 16, 16)]
                c_rel = jnp.right_shift(iv, 7) - lo_c
                m = jnp.logical_and(
                    jnp.logical_and(c_rel >= clo, c_rel < chi),
                    g * 16 + iota < cround_v)
                mi = m.astype(jnp.int32)
                rank = tot + plsc.cumsum(mi) - mi
                plsc.store_scatter(
                    ord_v, [jnp.clip(rank, 0, OCAP - 1)], slots, mask=m)
                return tot + plsc.all_reduce_population_count(m)

            return lax.fori_loop(0, ngrp, bkt, off_splat)

        off_splat = lax.fori_loop(0, NWIN, bktw, zeroi)
        offs_s[NWIN] = lax.reduce_max(off_splat, (0,))

        def rw_body(rw, carry):
            pltpu.make_async_copy(
                itabT.at[pl.ds(0, 8), pl.ds(0, WIN)],
                win_v.at[0], semw).wait()

            w, r8, serve, cst = win_params(rw)
            buf = lax.rem(rw, DEPTH)
            o_lo = offs_s[w]
            o_hi = offs_s[w + 1]
            glo = o_lo // 16
            ghi = (o_hi + 15) // 16
            olo = jnp.full((16,), o_lo, jnp.int32)
            ohi = jnp.full((16,), o_hi, jnp.int32)
            bufv = jnp.full((16,), buf, jnp.int32)
            servev = jnp.full((16,), serve, jnp.int32)
            cstv = jnp.full((16,), cst, jnp.int32)
            r8v = jnp.full((16,), r8, jnp.int32)

            def grp2(g, carry2):
                pos = g * 16 + iota
                slots = ord_v[pl.ds(g * 16, 16)]
                iv = plsc.load_gather(ii_v, [slots])
                icol = iv - cstv
                valid = jnp.logical_and(
                    jnp.logical_and(pos >= olo, pos < ohi),
                    jnp.logical_and(iv >= servev, icol < WIN))
                icl = jnp.clip(icol, 0, WIN - 1)
                a = zero
                for jj in range(8):
                    jv = jnp.full((16,), jj, jnp.int32)
                    v = plsc.load_gather(win_v, [bufv, jv, icl])
                    u = plsc.load_gather(uall_v, [r8v + jv, slots])
                    a = a + jnp.where(
                        valid, jnp.maximum(v, 0.0) * u, 0.0)
                prev = plsc.load_gather(acc_v, [slots])
                plsc.store_scatter(acc_v, [slots], prev + a, mask=valid)
                return carry2

            lax.fori_loop(glo, ghi, grp2, 0)

            @pl.when(rw + DEPTH < 8 * NWIN)
            def _():
                start_win(rw + DEPTH)

            return carry

        lax.fori_loop(0, 8 * NWIN, rw_body, 0)

        @pl.when(wid == NW - 1)
        def _():
            def tailr(r, carry):
                pltpu.sync_copy(
                    itabT.at[pl.ds(r * 8, 8), pl.ds(TAILSTART, TAILW)],
                    tail_v)
                r8v = jnp.full((16,), r * 8, jnp.int32)

                def grp3(g, carry2):
                    e0 = g * 16
                    iv = ii_v[pl.ds(e0, 16)]
                    icol = iv - TAILSTART
                    valid = jnp.logical_and(
                        jnp.logical_and(icol >= 0, icol < TAILW),
                        e0 + iota < cround_v)
                    icl = jnp.clip(icol, 0, TAILW - 1)
                    a = zero
                    for jj in range(8):
                        jv = jnp.full((16,), jj, jnp.int32)
                        v = plsc.load_gather(tail_v, [jv, icl])
                        u = plsc.load_gather(uall_v, [r8v + jv, e0 + iota])
                        a = a + jnp.where(
                            valid, jnp.maximum(v, 0.0) * u, 0.0)
                    acc_v[pl.ds(e0, 16)] = acc_v[pl.ds(e0, 16)] + a
                    return carry2

                lax.fori_loop(0, ngrp, grp3, 0)
                return carry

            lax.fori_loop(0, 8, tailr, 0)

        def sgrp(g, carry):
            e0 = g * 16
            bo = bo_v[g // 8, pl.ds((g % 8) * 16, 16)]
            ok = e0 + iota < cround_v
            plsc.store_scatter(
                outb_v,
                [jnp.right_shift(bo, 7), jnp.bitwise_and(bo, 127)],
                acc_v[pl.ds(e0, 16)], mask=ok)
            return carry

        lax.fori_loop(0, ngrp, sgrp, 0)

    def cond(state):
        rnd, total = state
        return rnd * OCAP < total

    def roundbody(state):
        rnd, _ = state
        for k in range(DEPTH):
            start_win(k)
        total = scan_fill(rnd)
        process(total, rnd)
        return rnd + 1, total

    lax.while_loop(cond, roundbody, (0, 1))

    pltpu.sync_copy(outb_v, shared_v.at[ridx_v], add=True)
    plsc.subcore_barrier()
    pltpu.sync_copy(
        shared_v.at[pl.ds(sid * 8, 8)],
        part_out.at[core, pl.ds(sid * 8, 8)])


def _body_merge(part_hbm, out_hbm, a_v, b_v, f_v, sem):
    wid = lax.axis_index("s") * NC + lax.axis_index("c")
    rows = BPW // 128
    pltpu.sync_copy(part_hbm.at[0, pl.ds(wid * rows, rows)], a_v)
    pltpu.sync_copy(part_hbm.at[1, pl.ds(wid * rows, rows)], b_v)

    def grp(g, carry):
        r = g // 8
        cc = (g % 8) * 16
        f_v[pl.ds(g * 16, 16)] = (
            a_v[r, pl.ds(cc, 16)] + b_v[r, pl.ds(cc, 16)])
        return carry

    lax.fori_loop(0, BPW // 16, grp, 0)
    pltpu.sync_copy(f_v, out_hbm.at[pl.ds(wid * BPW, BPW)])


@functools.partial(jax.jit, static_argnums=())
def _run(uidx, iidx, user_table, item_table):
    mesh = plsc.VectorSubcoreMesh(core_axis_name="c", subcore_axis_name="s")
    cp = pltpu.CompilerParams(
        needs_layout_passes=False, use_tc_tiling_on_sc=True,
        disable_bounds_checks=True)

    uidx2d = uidx.reshape(B // CHUNK, CHUNK)
    utab2 = user_table.reshape(NUM_USERS // 2, 2 * D)
    itabT = item_table.T

    ku = pl.kernel(
        _body_user, mesh=mesh,
        out_type=jax.ShapeDtypeStruct((B, 2 * D), jnp.float32),
        scratch_types=[
            pltpu.VMEM((BPW // CHUNK, CHUNK), jnp.int32),
            pltpu.VMEM((BPW // CHUNK, CHUNK), jnp.int32),
            pltpu.VMEM((BPW, 2 * D), jnp.float32),
            pltpu.SemaphoreType.DMA,
        ],
        compiler_params=cp)
    up = ku(uidx2d, utab2)

    ki = pl.kernel(
        _body_item, mesh=mesh,
        out_type=jax.ShapeDtypeStruct((NC, B // 128, 128), jnp.float32),
        scratch_types=[
            pltpu.VMEM((DEPTH, 8, WIN), jnp.float32),
            pltpu.VMEM((8, TAILW), jnp.float32),
            pltpu.VMEM((CHUNK, 2 * D), jnp.float32),
            pltpu.VMEM((D, OCAP), jnp.float32),
            pltpu.VMEM((B // 128, 128), jnp.float32),
            pltpu.VMEM((OCAP,), jnp.float32),
            pltpu.VMEM((OCAP,), jnp.int32),
            pltpu.VMEM((OCAP,), jnp.int32),
            pltpu.VMEM((OCAP // CHUNK, CHUNK), jnp.int32),
            pltpu.VMEM((OCAP,), jnp.int32),
            pltpu.VMEM((BLK,), jnp.int32),
            pltpu.VMEM((BLK,), jnp.int32),
            pltpu.VMEM((B // 128,), jnp.int32),
            pltpu.SMEM((NWIN + 1,), jnp.int32),
            pltpu.VMEM_SHARED((B // 128, 128), jnp.float32),
            pltpu.SemaphoreType.DMA,
            pltpu.SemaphoreType.DMA,
        ],
        compiler_params=cp)
    part = ki(uidx, iidx, itabT, up)

    km = pl.kernel(
        _body_merge, mesh=mesh,
        out_type=jax.ShapeDtypeStruct((B,), jnp.float32),
        scratch_types=[
            pltpu.VMEM((BPW // 128, 128), jnp.float32),
            pltpu.VMEM((BPW // 128, 128), jnp.float32),
            pltpu.VMEM((BPW,), jnp.float32),
            pltpu.SemaphoreType.DMA,
        ],
        compiler_params=cp)
    return km(part)


def kernel(user_indices, item_indices, user_table, item_table):
    return _run(user_indices.astype(jnp.int32),
                item_indices.astype(jnp.int32),
                user_table, item_table)


# speedup vs baseline: 2.8036x; 1.0109x over previous
import functools

import jax
import jax.numpy as jnp
from jax import lax
from jax.experimental import pallas as pl
from jax.experimental.pallas import tpu as pltpu
from jax.experimental.pallas import tpu_sc as plsc

NUM_USERS = 100000
NUM_ITEMS = 1000000
D = 64
B = 16384

NC = 2
NS = 16
NW = NC * NS
BPW = B // NW
CHUNK = 128
RANGE = NUM_ITEMS // NW
OCAP = 640
CW = 12
DEPTH = 4
WIN = CW * 128
NWIN = 21
CTILES = NUM_ITEMS // 128
LASTWINSTART = (CTILES - CW) * 128
TAILSTART = CTILES * 128
TAILW = NUM_ITEMS - TAILSTART
BLK = 512


def _body_user(uidx2d, utab2, up_out, uidx_v, keys_v, rows_v, sem):
    wid = lax.axis_index("s") * NC + lax.axis_index("c")
    pltpu.sync_copy(uidx2d.at[pl.ds(wid * (BPW // CHUNK), BPW // CHUNK)],
                    uidx_v)

    def keys(s, carry):
        c = s // 8
        g = s % 8
        u = uidx_v[c, pl.ds(g * 16, 16)]
        keys_v[c, pl.ds(g * 16, 16)] = jnp.right_shift(u, 1)
        return carry

    lax.fori_loop(0, (BPW // CHUNK) * 8, keys, 0)

    copies = []
    for c in range(BPW // CHUNK):
        copies.append(pltpu.async_copy(
            utab2.at[keys_v.at[c]],
            rows_v.at[pl.ds(c * CHUNK, CHUNK)], sem))
    for cp in copies:
        cp.wait()
    pltpu.sync_copy(rows_v, up_out.at[pl.ds(wid * BPW, BPW)])


def _body_item(uidx_hbm, iidx_hbm, itabT, utab2, part_out,
               win_v, tail_v, uch_v, uall_v, outb_v, acc_v,
               ii_v, uo_v, bo_v, ko_v, ord_v, blki_v, blku_v, ridx_v,
               offs_s, shared_v, sem, semw):
    core = lax.axis_index("c")
    sid = lax.axis_index("s")
    wid = sid * NC + core
    lo = wid * RANGE
    hi = lo + RANGE
    lo_c = lo // 128
    iota = lax.iota(jnp.int32, 16)
    zero = jnp.zeros((16,), jnp.float32)
    zeroi = jnp.zeros((16,), jnp.int32)

    def zout(g, carry):
        outb_v[g // 8, pl.ds((g % 8) * 16, 16)] = zero
        return carry

    lax.fori_loop(0, B // 16, zout, 0)

    def zridx(g, carry):
        ridx_v[pl.ds(g * 16, 16)] = g * 16 + iota
        return carry

    lax.fori_loop(0, (B // 128) // 16, zridx, 0)

    @pl.when(sid == 0)
    def _():
        pltpu.sync_copy(outb_v, shared_v)

    plsc.subcore_barrier()

    def win_params(rw):
        w = lax.rem(rw, NWIN)
        r8 = (rw // NWIN) * 8
        serve = (lo_c + w * CW) * 128
        cst = pl.multiple_of(jnp.minimum(serve, LASTWINSTART), 128)
        return w, r8, serve, cst

    def start_win(rw):
        w, r8, serve, cst = win_params(rw)
        pltpu.async_copy(
            itabT.at[pl.ds(r8, 8), pl.ds(cst, WIN)],
            win_v.at[lax.rem(rw, DEPTH)], semw)

    def scan_fill(rnd):
        rlo = rnd * OCAP

        def zlists(g, carry):
            ii_v[pl.ds(g * 16, 16)] = zeroi
            uo_v[pl.ds(g * 16, 16)] = zeroi
            bo_v[g // 8, pl.ds((g % 8) * 16, 16)] = zeroi
            ko_v[g // 8, pl.ds((g % 8) * 16, 16)] = zeroi
            ord_v[pl.ds(g * 16, 16)] = zeroi
            return carry

        lax.fori_loop(0, OCAP // 16, zlists, 0)

        def blk_body(blk, tot):
            c1 = pltpu.async_copy(
                iidx_hbm.at[pl.ds(blk * BLK, BLK)], blki_v, sem)
            c2 = pltpu.async_copy(
                uidx_hbm.at[pl.ds(blk * BLK, BLK)], blku_v, sem)
            c1.wait()
            c2.wait()

            def step(s, tot):
                iv = blki_v[pl.ds(s * 16, 16)]
                uv = blku_v[pl.ds(s * 16, 16)]
                m = jnp.logical_and(iv >= lo, iv < hi)
                mi = m.astype(jnp.int32)
                rank = tot + plsc.cumsum(mi) - mi
                slot = rank - rlo
                keep = jnp.logical_and(
                    m, jnp.logical_and(slot >= 0, slot < OCAP))
                slot_c = jnp.clip(slot, 0, OCAP - 1)
                bvec = blk * BLK + s * 16 + iota
                plsc.store_scatter(ii_v, [slot_c], iv, mask=keep)
                plsc.store_scatter(uo_v, [slot_c], uv, mask=keep)
                plsc.store_scatter(
                    bo_v,
                    [jnp.right_shift(slot_c, 7),
                     jnp.bitwise_and(slot_c, 127)],
                    bvec, mask=keep)
                cnt = plsc.all_reduce_population_count(m)
                return tot + cnt

            return lax.fori_loop(0, BLK // 16, step, tot)

        tot = lax.fori_loop(0, B // BLK, blk_body, zeroi)
        return lax.reduce_max(tot, (0,))

    def process(total, rnd):
        cround = jnp.clip(total - rnd * OCAP, 0, OCAP)
        nch = (cround + CHUNK - 1) // CHUNK
        ngrp = (cround + 15) // 16
        cround_v = jnp.full((16,), cround, jnp.int32)

        def zacc(g, carry):
            acc_v[pl.ds(g * 16, 16)] = zero
            return carry

        lax.fori_loop(0, OCAP // 16, zacc, 0)

        def kg(g, carry):
            ko_v[g // 8, pl.ds((g % 8) * 16, 16)] = jnp.right_shift(
                uo_v[pl.ds(g * 16, 16)], 1)
            return carry

        lax.fori_loop(0, OCAP // 16, kg, 0)

        def chb(ch, carry):
            pltpu.async_copy(utab2.at[ko_v.at[ch]], uch_v, sem).wait()

            def grp(g, carry2):
                rows = g * 16 + iota
                uo = uo_v[pl.ds(ch * CHUNK + g * 16, 16)]
                uoff = jnp.left_shift(jnp.bitwise_and(uo, 1), 6)
                for d in range(D):
                    u = plsc.load_gather(uch_v, [rows, uoff + d])
                    plsc.store_scatter(
                        uall_v, [jnp.full((16,), d, jnp.int32),
                                 ch * CHUNK + rows],
                        jnp.maximum(u, 0.0))
                return carry2

            lax.fori_loop(0, 8, grp, 0)
            return carry

        lax.fori_loop(0, nch, chb, 0)

        def bktw(w, off_splat):
            offs_s[w] = lax.reduce_max(off_splat, (0,))
            clo = jnp.full((16,), w * CW, jnp.int32)
            chi = jnp.full((16,), (w + 1) * CW, jnp.int32)

            def bkt(g, tot):
                slots = g * 16 + iota
                iv = ii_v[pl.ds(g * 16, 16)]
                c_rel = jnp.right_shift(iv, 7) - lo_c
                m = jnp.logical_and(
                    jnp.logical_and(c_rel >= clo, c_rel < chi),
                    g * 16 + iota < cround_v)
                mi = m.astype(jnp.int32)
                rank = tot + plsc.cumsum(mi) - mi
                plsc.store_scatter(
                    ord_v, [jnp.clip(rank, 0, OCAP - 1)], slots, mask=m)
                return tot + plsc.all_reduce_population_count(m)

            return lax.fori_loop(0, ngrp, bkt, off_splat)

        off_splat = lax.fori_loop(0, NWIN, bktw, zeroi)
        offs_s[NWIN] = lax.reduce_max(off_splat, (0,))

        def rw_body(rw, carry):
            pltpu.make_async_copy(
                itabT.at[pl.ds(0, 8), pl.ds(0, WIN)],
                win_v.at[0], semw).wait()

            w, r8, serve, cst = win_params(rw)
            buf = lax.rem(rw, DEPTH)
            o_lo = offs_s[w]
            o_hi = offs_s[w + 1]
            glo = o_lo // 16
            ghi = (o_hi + 15) // 16
            olo = jnp.full((16,), o_lo, jnp.int32)
            ohi = jnp.full((16,), o_hi, jnp.int32)
            bufv = jnp.full((16,), buf, jnp.int32)
            servev = jnp.full((16,), serve, jnp.int32)
            cstv = jnp.full((16,), cst, jnp.int32)
            r8v = jnp.full((16,), r8, jnp.int32)

            def grp2(g, carry2):
                pos = g * 16 + iota
                slots = ord_v[pl.ds(g * 16, 16)]
                iv = plsc.load_gather(ii_v, [slots])
                icol = iv - cstv
                valid = jnp.logical_and(
                    jnp.logical_and(pos >= olo, pos < ohi),
                    jnp.logical_and(iv >= servev, icol < WIN))
                icl = jnp.clip(icol, 0, WIN - 1)
                a = zero
                for jj in range(8):
                    jv = jnp.full((16,), jj, jnp.int32)
                    v = plsc.load_gather(win_v, [bufv, jv, icl])
                    u = plsc.load_gather(uall_v, [r8v + jv, slots])
                    a = a + jnp.where(
                        valid, jnp.maximum(v, 0.0) * u, 0.0)
                prev = plsc.load_gather(acc_v, [slots])
                plsc.store_scatter(acc_v, [slots], prev + a, mask=valid)
                return carry2

            lax.fori_loop(glo, ghi, grp2, 0)

            @pl.when(rw + DEPTH < 8 * NWIN)
            def _():
                start_win(rw + DEPTH)

            return carry

        lax.fori_loop(0, 8 * NWIN, rw_body, 0)

        @pl.when(wid == NW - 1)
        def _():
            def tailr(r, carry):
                pltpu.sync_copy(
                    itabT.at[pl.ds(r * 8, 8), pl.ds(TAILSTART, TAILW)],
                    tail_v)
                r8v = jnp.full((16,), r * 8, jnp.int32)

                def grp3(g, carry2):
                    e0 = g * 16
                    iv = ii_v[pl.ds(e0, 16)]
                    icol = iv - TAILSTART
                    valid = jnp.logical_and(
                        jnp.logical_and(icol >= 0, icol < TAILW),
                        e0 + iota < cround_v)
                    icl = jnp.clip(icol, 0, TAILW - 1)
                    a = zero
                    for jj in range(8):
                        jv = jnp.full((16,), jj, jnp.int32)
                        v = plsc.load_gather(tail_v, [jv, icl])
                        u = plsc.load_gather(uall_v, [r8v + jv, e0 + iota])
                        a = a + jnp.where(
                            valid, jnp.maximum(v, 0.0) * u, 0.0)
                    acc_v[pl.ds(e0, 16)] = acc_v[pl.ds(e0, 16)] + a
                    return carry2

                lax.fori_loop(0, ngrp, grp3, 0)
                return carry

            lax.fori_loop(0, 8, tailr, 0)

        def sgrp(g, carry):
            e0 = g * 16
            bo = bo_v[g // 8, pl.ds((g % 8) * 16, 16)]
            ok = e0 + iota < cround_v
            plsc.store_scatter(
                outb_v,
                [jnp.right_shift(bo, 7), jnp.bitwise_and(bo, 127)],
                acc_v[pl.ds(e0, 16)], mask=ok)
            return carry

        lax.fori_loop(0, ngrp, sgrp, 0)

    def cond(state):
        rnd, total = state
        return rnd * OCAP < total

    def roundbody(state):
        rnd, _ = state
        for k in range(DEPTH):
            start_win(k)
        total = scan_fill(rnd)
        process(total, rnd)
        return rnd + 1, total

    lax.while_loop(cond, roundbody, (0, 1))

    pltpu.sync_copy(outb_v, shared_v.at[ridx_v], add=True)
    plsc.subcore_barrier()
    pltpu.sync_copy(
        shared_v.at[pl.ds(sid * 8, 8)],
        part_out.at[core, pl.ds(sid * 8, 8)])


def _body_merge(part_hbm, out_hbm, a_v, b_v, f_v, sem):
    wid = lax.axis_index("s") * NC + lax.axis_index("c")
    rows = BPW // 128
    pltpu.sync_copy(part_hbm.at[0, pl.ds(wid * rows, rows)], a_v)
    pltpu.sync_copy(part_hbm.at[1, pl.ds(wid * rows, rows)], b_v)

    def grp(g, carry):
        r = g // 8
        cc = (g % 8) * 16
        f_v[pl.ds(g * 16, 16)] = (
            a_v[r, pl.ds(cc, 16)] + b_v[r, pl.ds(cc, 16)])
        return carry

    lax.fori_loop(0, BPW // 16, grp, 0)
    pltpu.sync_copy(f_v, out_hbm.at[pl.ds(wid * BPW, BPW)])


@functools.partial(jax.jit, static_argnums=())
def _run(uidx, iidx, user_table, item_table):
    mesh = plsc.VectorSubcoreMesh(core_axis_name="c", subcore_axis_name="s")
    cp = pltpu.CompilerParams(
        needs_layout_passes=False, use_tc_tiling_on_sc=True,
        disable_bounds_checks=True)

    uidx2d = uidx.reshape(B // CHUNK, CHUNK)
    utab2 = user_table.reshape(NUM_USERS // 2, 2 * D)
    itabT = item_table.T

    ki = pl.kernel(
        _body_item, mesh=mesh,
        out_type=jax.ShapeDtypeStruct((NC, B // 128, 128), jnp.float32),
        scratch_types=[
            pltpu.VMEM((DEPTH, 8, WIN), jnp.float32),
            pltpu.VMEM((8, TAILW), jnp.float32),
            pltpu.VMEM((CHUNK, 2 * D), jnp.float32),
            pltpu.VMEM((D, OCAP), jnp.float32),
            pltpu.VMEM((B // 128, 128), jnp.float32),
            pltpu.VMEM((OCAP,), jnp.float32),
            pltpu.VMEM((OCAP,), jnp.int32),
            pltpu.VMEM((OCAP,), jnp.int32),
            pltpu.VMEM((OCAP // CHUNK, CHUNK), jnp.int32),
            pltpu.VMEM((OCAP // CHUNK, CHUNK), jnp.int32),
            pltpu.VMEM((OCAP,), jnp.int32),
            pltpu.VMEM((BLK,), jnp.int32),
            pltpu.VMEM((BLK,), jnp.int32),
            pltpu.VMEM((B // 128,), jnp.int32),
            pltpu.SMEM((NWIN + 1,), jnp.int32),
            pltpu.VMEM_SHARED((B // 128, 128), jnp.float32),
            pltpu.SemaphoreType.DMA,
            pltpu.SemaphoreType.DMA,
        ],
        compiler_params=cp)
    part = ki(uidx, iidx, itabT, utab2)

    km = pl.kernel(
        _body_merge, mesh=mesh,
        out_type=jax.ShapeDtypeStruct((B,), jnp.float32),
        scratch_types=[
            pltpu.VMEM((BPW // 128, 128), jnp.float32),
            pltpu.VMEM((BPW // 128, 128), jnp.float32),
            pltpu.VMEM((BPW,), jnp.float32),
            pltpu.SemaphoreType.DMA,
        ],
        compiler_params=cp)
    return km(part)


def kernel(user_indices, item_indices, user_table, item_table):
    return _run(user_indices.astype(jnp.int32),
                item_indices.astype(jnp.int32),
                user_table, item_table)


# speedup vs baseline: 2.8191x; 1.0055x over previous
import functools

import jax
import jax.numpy as jnp
from jax import lax
from jax.experimental import pallas as pl
from jax.experimental.pallas import tpu as pltpu
from jax.experimental.pallas import tpu_sc as plsc

NUM_USERS = 100000
NUM_ITEMS = 1000000
D = 64
B = 16384

NC = 2
NS = 16
NW = NC * NS
BPW = B // NW
CHUNK = 128
RANGE = NUM_ITEMS // NW
OCAP = 640
CW = 12
DEPTH = 4
WIN = CW * 128
NWIN = 21
CTILES = NUM_ITEMS // 128
LASTWINSTART = (CTILES - CW) * 128
TAILSTART = CTILES * 128
TAILW = NUM_ITEMS - TAILSTART
BLK = 512


def _body_user(uidx2d, utab2, up_out, uidx_v, keys_v, rows_v, sem):
    wid = lax.axis_index("s") * NC + lax.axis_index("c")
    pltpu.sync_copy(uidx2d.at[pl.ds(wid * (BPW // CHUNK), BPW // CHUNK)],
                    uidx_v)

    def keys(s, carry):
        c = s // 8
        g = s % 8
        u = uidx_v[c, pl.ds(g * 16, 16)]
        keys_v[c, pl.ds(g * 16, 16)] = jnp.right_shift(u, 1)
        return carry

    lax.fori_loop(0, (BPW // CHUNK) * 8, keys, 0)

    copies = []
    for c in range(BPW // CHUNK):
        copies.append(pltpu.async_copy(
            utab2.at[keys_v.at[c]],
            rows_v.at[pl.ds(c * CHUNK, CHUNK)], sem))
    for cp in copies:
        cp.wait()
    pltpu.sync_copy(rows_v, up_out.at[pl.ds(wid * BPW, BPW)])


def _body_item(uidx_hbm, iidx_hbm, itabT, utab2, part_out,
               win_v, tail_v, uch_v, uall_v, outb_v, acc_v,
               ii_v, uo_v, bo_v, ko_v, ord_v, blki_v, blku_v,
               offs_s, shared_v, sem, semw):
    core = lax.axis_index("c")
    sid = lax.axis_index("s")
    wid = sid * NC + core
    lo = wid * RANGE
    hi = lo + RANGE
    lo_c = lo // 128
    iota = lax.iota(jnp.int32, 16)
    zero = jnp.zeros((16,), jnp.float32)
    zeroi = jnp.zeros((16,), jnp.int32)

    def zout(g, carry):
        outb_v[pl.ds(g * 16, 16)] = zero
        return carry

    lax.fori_loop(0, B // 16, zout, 0)

    @pl.when(sid == 0)
    def _():
        pltpu.sync_copy(outb_v, shared_v)

    plsc.subcore_barrier()

    def win_params(rw):
        w = lax.rem(rw, NWIN)
        r8 = (rw // NWIN) * 8
        serve = (lo_c + w * CW) * 128
        cst = pl.multiple_of(jnp.minimum(serve, LASTWINSTART), 128)
        return w, r8, serve, cst

    def start_win(rw):
        w, r8, serve, cst = win_params(rw)
        pltpu.async_copy(
            itabT.at[pl.ds(r8, 8), pl.ds(cst, WIN)],
            win_v.at[lax.rem(rw, DEPTH)], semw)

    def scan_fill(rnd):
        rlo = rnd * OCAP

        def zlists(g, carry):
            ii_v[pl.ds(g * 16, 16)] = zeroi
            uo_v[pl.ds(g * 16, 16)] = zeroi
            bo_v[g // 8, pl.ds((g % 8) * 16, 16)] = zeroi
            ko_v[g // 8, pl.ds((g % 8) * 16, 16)] = zeroi
            ord_v[pl.ds(g * 16, 16)] = zeroi
            return carry

        lax.fori_loop(0, OCAP // 16, zlists, 0)

        def blk_body(blk, tot):
            c1 = pltpu.async_copy(
                iidx_hbm.at[pl.ds(blk * BLK, BLK)], blki_v, sem)
            c2 = pltpu.async_copy(
                uidx_hbm.at[pl.ds(blk * BLK, BLK)], blku_v, sem)
            c1.wait()
            c2.wait()

            def step(s, tot):
                iv = blki_v[pl.ds(s * 16, 16)]
                uv = blku_v[pl.ds(s * 16, 16)]
                m = jnp.logical_and(iv >= lo, iv < hi)
                mi = m.astype(jnp.int32)
                rank = tot + plsc.cumsum(mi) - mi
                slot = rank - rlo
                keep = jnp.logical_and(
                    m, jnp.logical_and(slot >= 0, slot < OCAP))
                slot_c = jnp.clip(slot, 0, OCAP - 1)
                bvec = blk * BLK + s * 16 + iota
                plsc.store_scatter(ii_v, [slot_c], iv, mask=keep)
                plsc.store_scatter(uo_v, [slot_c], uv, mask=keep)
                plsc.store_scatter(
                    bo_v,
                    [jnp.right_shift(slot_c, 7),
                     jnp.bitwise_and(slot_c, 127)],
                    bvec, mask=keep)
                cnt = plsc.all_reduce_population_count(m)
                return tot + cnt

            return lax.fori_loop(0, BLK // 16, step, tot)

        tot = lax.fori_loop(0, B // BLK, blk_body, zeroi)
        return lax.reduce_max(tot, (0,))

    def process(total, rnd):
        cround = jnp.clip(total - rnd * OCAP, 0, OCAP)
        nch = (cround + CHUNK - 1) // CHUNK
        ngrp = (cround + 15) // 16
        cround_v = jnp.full((16,), cround, jnp.int32)

        def zacc(g, carry):
            acc_v[pl.ds(g * 16, 16)] = zero
            return carry

        lax.fori_loop(0, OCAP // 16, zacc, 0)

        def kg(g, carry):
            ko_v[g // 8, pl.ds((g % 8) * 16, 16)] = jnp.right_shift(
                uo_v[pl.ds(g * 16, 16)], 1)
            return carry

        lax.fori_loop(0, OCAP // 16, kg, 0)

        def chb(ch, carry):
            pltpu.async_copy(utab2.at[ko_v.at[ch]], uch_v, sem).wait()

            def grp(g, carry2):
                rows = g * 16 + iota
                uo = uo_v[pl.ds(ch * CHUNK + g * 16, 16)]
                uoff = jnp.left_shift(jnp.bitwise_and(uo, 1), 6)
                for d in range(D):
                    u = plsc.load_gather(uch_v, [rows, uoff + d])
                    plsc.store_scatter(
                        uall_v, [jnp.full((16,), d, jnp.int32),
                                 ch * CHUNK + rows],
                        jnp.maximum(u, 0.0))
                return carry2

            lax.fori_loop(0, 8, grp, 0)
            return carry

        lax.fori_loop(0, nch, chb, 0)

        def bktw(w, off_splat):
            offs_s[w] = lax.reduce_max(off_splat, (0,))
            clo = jnp.full((16,), w * CW, jnp.int32)
            chi = jnp.full((16,), (w + 1) * CW, jnp.int32)

            def bkt(g, tot):
                slots = g * 16 + iota
                iv = ii_v[pl.ds(g * 16, 16)]
                c_rel = jnp.right_shift(iv, 7) - lo_c
                m = jnp.logical_and(
                    jnp.logical_and(c_rel >= clo, c_rel < chi),
                    g * 16 + iota < cround_v)
                mi = m.astype(jnp.int32)
                rank = tot + plsc.cumsum(mi) - mi
                plsc.store_scatter(
                    ord_v, [jnp.clip(rank, 0, OCAP - 1)], slots, mask=m)
                return tot + plsc.all_reduce_population_count(m)

            return lax.fori_loop(0, ngrp, bkt, off_splat)

        off_splat = lax.fori_loop(0, NWIN, bktw, zeroi)
        offs_s[NWIN] = lax.reduce_max(off_splat, (0,))

        def rw_body(rw, carry):
            pltpu.make_async_copy(
                itabT.at[pl.ds(0, 8), pl.ds(0, WIN)],
                win_v.at[0], semw).wait()

            w, r8, serve, cst = win_params(rw)
            buf = lax.rem(rw, DEPTH)
            o_lo = offs_s[w]
            o_hi = offs_s[w + 1]
            glo = o_lo // 16
            ghi = (o_hi + 15) // 16
            olo = jnp.full((16,), o_lo, jnp.int32)
            ohi = jnp.full((16,), o_hi, jnp.int32)
            bufv = jnp.full((16,), buf, jnp.int32)
            servev = jnp.full((16,), serve, jnp.int32)
            cstv = jnp.full((16,), cst, jnp.int32)
            r8v = jnp.full((16,), r8, jnp.int32)

            def grp2(g, carry2):
                pos = g * 16 + iota
                slots = ord_v[pl.ds(g * 16, 16)]
                iv = plsc.load_gather(ii_v, [slots])
                icol = iv - cstv
                valid = jnp.logical_and(
                    jnp.logical_and(pos >= olo, pos < ohi),
                    jnp.logical_and(iv >= servev, icol < WIN))
                icl = jnp.clip(icol, 0, WIN - 1)
                a = zero
                for jj in range(8):
                    jv = jnp.full((16,), jj, jnp.int32)
                    v = plsc.load_gather(win_v, [bufv, jv, icl])
                    u = plsc.load_gather(uall_v, [r8v + jv, slots])
                    a = a + jnp.where(
                        valid, jnp.maximum(v, 0.0) * u, 0.0)
                prev = plsc.load_gather(acc_v, [slots])
                plsc.store_scatter(acc_v, [slots], prev + a, mask=valid)
                return carry2

            lax.fori_loop(glo, ghi, grp2, 0)

            @pl.when(rw + DEPTH < 8 * NWIN)
            def _():
                start_win(rw + DEPTH)

            return carry

        lax.fori_loop(0, 8 * NWIN, rw_body, 0)

        @pl.when(wid == NW - 1)
        def _():
            def tailr(r, carry):
                pltpu.sync_copy(
                    itabT.at[pl.ds(r * 8, 8), pl.ds(TAILSTART, TAILW)],
                    tail_v)
                r8v = jnp.full((16,), r * 8, jnp.int32)

                def grp3(g, carry2):
                    e0 = g * 16
                    iv = ii_v[pl.ds(e0, 16)]
                    icol = iv - TAILSTART
                    valid = jnp.logical_and(
                        jnp.logical_and(icol >= 0, icol < TAILW),
                        e0 + iota < cround_v)
                    icl = jnp.clip(icol, 0, TAILW - 1)
                    a = zero
                    for jj in range(8):
                        jv = jnp.full((16,), jj, jnp.int32)
                        v = plsc.load_gather(tail_v, [jv, icl])
                        u = plsc.load_gather(uall_v, [r8v + jv, e0 + iota])
                        a = a + jnp.where(
                            valid, jnp.maximum(v, 0.0) * u, 0.0)
                    acc_v[pl.ds(e0, 16)] = acc_v[pl.ds(e0, 16)] + a
                    return carry2

                lax.fori_loop(0, ngrp, grp3, 0)
                return carry

            lax.fori_loop(0, 8, tailr, 0)

        def sch(ch, carry):
            pltpu.sync_copy(
                acc_v.at[pl.ds(ch * CHUNK, CHUNK)],
                shared_v.at[bo_v.at[ch]], add=True)
            return carry

        lax.fori_loop(0, nch, sch, 0)

    def cond(state):
        rnd, total = state
        return rnd * OCAP < total

    def roundbody(state):
        rnd, _ = state
        for k in range(DEPTH):
            start_win(k)
        total = scan_fill(rnd)
        process(total, rnd)
        return rnd + 1, total

    lax.while_loop(cond, roundbody, (0, 1))

    plsc.subcore_barrier()
    pltpu.sync_copy(
        shared_v.at[pl.ds(sid * (B // NS), B // NS)],
        part_out.at[core, pl.ds(sid * (B // NS), B // NS)])


def _body_merge(part_hbm, out_hbm, a_v, b_v, f_v, sem):
    wid = lax.axis_index("s") * NC + lax.axis_index("c")
    base = wid * BPW
    pltpu.sync_copy(part_hbm.at[0, pl.ds(base, BPW)], a_v)
    pltpu.sync_copy(part_hbm.at[1, pl.ds(base, BPW)], b_v)

    def grp(g, carry):
        f_v[pl.ds(g * 16, 16)] = (
            a_v[pl.ds(g * 16, 16)] + b_v[pl.ds(g * 16, 16)])
        return carry

    lax.fori_loop(0, BPW // 16, grp, 0)
    pltpu.sync_copy(f_v, out_hbm.at[pl.ds(base, BPW)])


@functools.partial(jax.jit, static_argnums=())
def _run(uidx, iidx, user_table, item_table):
    mesh = plsc.VectorSubcoreMesh(core_axis_name="c", subcore_axis_name="s")
    cp = pltpu.CompilerParams(
        needs_layout_passes=False, use_tc_tiling_on_sc=True,
        disable_bounds_checks=True)

    uidx2d = uidx.reshape(B // CHUNK, CHUNK)
    utab2 = user_table.reshape(NUM_USERS // 2, 2 * D)
    itabT = item_table.T

    ki = pl.kernel(
        _body_item, mesh=mesh,
        out_type=jax.ShapeDtypeStruct((NC, B), jnp.float32),
        scratch_types=[
            pltpu.VMEM((DEPTH, 8, WIN), jnp.float32),
            pltpu.VMEM((8, TAILW), jnp.float32),
            pltpu.VMEM((CHUNK, 2 * D), jnp.float32),
            pltpu.VMEM((D, OCAP), jnp.float32),
            pltpu.VMEM((B,), jnp.float32),
            pltpu.VMEM((OCAP,), jnp.float32),
            pltpu.VMEM((OCAP,), jnp.int32),
            pltpu.VMEM((OCAP,), jnp.int32),
            pltpu.VMEM((OCAP // CHUNK, CHUNK), jnp.int32),
            pltpu.VMEM((OCAP // CHUNK, CHUNK), jnp.int32),
            pltpu.VMEM((OCAP,), jnp.int32),
            pltpu.VMEM((BLK,), jnp.int32),
            pltpu.VMEM((BLK,), jnp.int32),
            pltpu.SMEM((NWIN + 1,), jnp.int32),
            pltpu.VMEM_SHARED((B,), jnp.float32),
            pltpu.SemaphoreType.DMA,
            pltpu.SemaphoreType.DMA,
        ],
        compiler_params=cp)
    part = ki(uidx, iidx, itabT, utab2)

    km = pl.kernel(
        _body_merge, mesh=mesh,
        out_type=jax.ShapeDtypeStruct((B,), jnp.float32),
        scratch_types=[
            pltpu.VMEM((BPW,), jnp.float32),
            pltpu.VMEM((BPW,), jnp.float32),
            pltpu.VMEM((BPW,), jnp.float32),
            pltpu.SemaphoreType.DMA,
        ],
        compiler_params=cp)
    return km(part)


def kernel(user_indices, item_indices, user_table, item_table):
    return _run(user_indices.astype(jnp.int32),
                item_indices.astype(jnp.int32),
                user_table, item_table)


# speedup vs baseline: 2.9629x; 1.0510x over previous
import functools

import jax
import jax.numpy as jnp
from jax import lax
from jax.experimental import pallas as pl
from jax.experimental.pallas import tpu as pltpu
from jax.experimental.pallas import tpu_sc as plsc

NUM_USERS = 100000
NUM_ITEMS = 1000000
D = 64
B = 16384

NC = 2
NS = 16
NW = NC * NS
BPW = B // NW
CHUNK = 128
RANGE = NUM_ITEMS // NW
OCAP = 640
CW = 12
DEPTH = 4
WIN = CW * 128
NWIN = 21
CTILES = NUM_ITEMS // 128
LASTWINSTART = (CTILES - CW) * 128
TAILSTART = CTILES * 128
TAILW = NUM_ITEMS - TAILSTART
BLK = 512


def _body_user(uidx2d, utab2, up_out, uidx_v, keys_v, rows_v, sem):
    wid = lax.axis_index("s") * NC + lax.axis_index("c")
    pltpu.sync_copy(uidx2d.at[pl.ds(wid * (BPW // CHUNK), BPW // CHUNK)],
                    uidx_v)

    def keys(s, carry):
        c = s // 8
        g = s % 8
        u = uidx_v[c, pl.ds(g * 16, 16)]
        keys_v[c, pl.ds(g * 16, 16)] = jnp.right_shift(u, 1)
        return carry

    lax.fori_loop(0, (BPW // CHUNK) * 8, keys, 0)

    copies = []
    for c in range(BPW // CHUNK):
        copies.append(pltpu.async_copy(
            utab2.at[keys_v.at[c]],
            rows_v.at[pl.ds(c * CHUNK, CHUNK)], sem))
    for cp in copies:
        cp.wait()
    pltpu.sync_copy(rows_v, up_out.at[pl.ds(wid * BPW, BPW)])


def _body_item(uidx_hbm, iidx_hbm, itabT, utab2, part_out,
               win_v, tail_v, uch_v, uall_v, zb_v, acc_v,
               ii_v, uo_v, bo_v, ko_v, ord_v, blki_v, blku_v,
               offs_s, shared_v, sem, semw):
    core = lax.axis_index("c")
    sid = lax.axis_index("s")
    wid = sid * NC + core
    lo = wid * RANGE
    hi = lo + RANGE
    lo_c = lo // 128
    iota = lax.iota(jnp.int32, 16)
    zero = jnp.zeros((16,), jnp.float32)
    zeroi = jnp.zeros((16,), jnp.int32)

    def zout(g, carry):
        zb_v[pl.ds(g * 16, 16)] = zero
        return carry

    lax.fori_loop(0, (B // NS) // 16, zout, 0)
    pltpu.sync_copy(zb_v, shared_v.at[pl.ds(sid * (B // NS), B // NS)])
    plsc.subcore_barrier()

    def win_params(rw):
        w = lax.rem(rw, NWIN)
        r8 = (rw // NWIN) * 8
        serve = (lo_c + w * CW) * 128
        cst = pl.multiple_of(jnp.minimum(serve, LASTWINSTART), 128)
        return w, r8, serve, cst

    def start_win(rw):
        w, r8, serve, cst = win_params(rw)
        pltpu.async_copy(
            itabT.at[pl.ds(r8, 8), pl.ds(cst, WIN)],
            win_v.at[lax.rem(rw, DEPTH)], semw)

    def scan_fill(rnd):
        rlo = rnd * OCAP

        def zlists(g, carry):
            ii_v[pl.ds(g * 16, 16)] = zeroi
            uo_v[pl.ds(g * 16, 16)] = zeroi
            bo_v[g // 8, pl.ds((g % 8) * 16, 16)] = zeroi
            ko_v[g // 8, pl.ds((g % 8) * 16, 16)] = zeroi
            ord_v[pl.ds(g * 16, 16)] = zeroi
            return carry

        lax.fori_loop(0, OCAP // 16, zlists, 0)

        def start_blk(blk):
            pltpu.async_copy(
                iidx_hbm.at[pl.ds(blk * BLK, BLK)],
                blki_v.at[lax.rem(blk, 2)], sem)
            pltpu.async_copy(
                uidx_hbm.at[pl.ds(blk * BLK, BLK)],
                blku_v.at[lax.rem(blk, 2)], sem)

        start_blk(0)

        def blk_body(blk, tot):
            pltpu.make_async_copy(
                iidx_hbm.at[pl.ds(0, BLK)], blki_v.at[0], sem).wait()
            pltpu.make_async_copy(
                uidx_hbm.at[pl.ds(0, BLK)], blku_v.at[0], sem).wait()

            @pl.when(blk + 1 < B // BLK)
            def _():
                start_blk(blk + 1)

            bbuf = lax.rem(blk, 2)

            def step(s, tot):
                iv = blki_v[bbuf, pl.ds(s * 16, 16)]
                uv = blku_v[bbuf, pl.ds(s * 16, 16)]
                m = jnp.logical_and(iv >= lo, iv < hi)
                mi = m.astype(jnp.int32)
                rank = tot + plsc.cumsum(mi) - mi
                slot = rank - rlo
                keep = jnp.logical_and(
                    m, jnp.logical_and(slot >= 0, slot < OCAP))
                slot_c = jnp.clip(slot, 0, OCAP - 1)
                bvec = blk * BLK + s * 16 + iota
                plsc.store_scatter(ii_v, [slot_c], iv, mask=keep)
                plsc.store_scatter(uo_v, [slot_c], uv, mask=keep)
                plsc.store_scatter(
                    bo_v,
                    [jnp.right_shift(slot_c, 7),
                     jnp.bitwise_and(slot_c, 127)],
                    bvec, mask=keep)
                cnt = plsc.all_reduce_population_count(m)
                return tot + cnt

            return lax.fori_loop(0, BLK // 16, step, tot)

        tot = lax.fori_loop(0, B // BLK, blk_body, zeroi)
        return lax.reduce_max(tot, (0,))

    def process(total, rnd):
        cround = jnp.clip(total - rnd * OCAP, 0, OCAP)
        nch = (cround + CHUNK - 1) // CHUNK
        ngrp = (cround + 15) // 16
        cround_v = jnp.full((16,), cround, jnp.int32)

        def zacc(g, carry):
            acc_v[pl.ds(g * 16, 16)] = zero
            return carry

        lax.fori_loop(0, OCAP // 16, zacc, 0)

        def kg(g, carry):
            ko_v[g // 8, pl.ds((g % 8) * 16, 16)] = jnp.right_shift(
                uo_v[pl.ds(g * 16, 16)], 1)
            return carry

        lax.fori_loop(0, OCAP // 16, kg, 0)

        def chb(ch, carry):
            pltpu.async_copy(utab2.at[ko_v.at[ch]], uch_v, sem).wait()

            def grp(g, carry2):
                rows = g * 16 + iota
                uo = uo_v[pl.ds(ch * CHUNK + g * 16, 16)]
                uoff = jnp.left_shift(jnp.bitwise_and(uo, 1), 6)
                for d in range(D):
                    u = plsc.load_gather(uch_v, [rows, uoff + d])
                    plsc.store_scatter(
                        uall_v, [jnp.full((16,), d, jnp.int32),
                                 ch * CHUNK + rows],
                        jnp.maximum(u, 0.0))
                return carry2

            lax.fori_loop(0, 8, grp, 0)
            return carry

        lax.fori_loop(0, nch, chb, 0)

        def bktw(w, off_splat):
            offs_s[w] = lax.reduce_max(off_splat, (0,))
            clo = jnp.full((16,), w * CW, jnp.int32)
            chi = jnp.full((16,), (w + 1) * CW, jnp.int32)

            def bkt(g, tot):
                slots = g * 16 + iota
                iv = ii_v[pl.ds(g * 16, 16)]
                c_rel = jnp.right_shift(iv, 7) - lo_c
                m = jnp.logical_and(
                    jnp.logical_and(c_rel >= clo, c_rel < chi),
                    g * 16 + iota < cround_v)
                mi = m.astype(jnp.int32)
                rank = tot + plsc.cumsum(mi) - mi
                plsc.store_scatter(
                    ord_v, [jnp.clip(rank, 0, OCAP - 1)], slots, mask=m)
                return tot + plsc.all_reduce_population_count(m)

            return lax.fori_loop(0, ngrp, bkt, off_splat)

        off_splat = lax.fori_loop(0, NWIN, bktw, zeroi)
        offs_s[NWIN] = lax.reduce_max(off_splat, (0,))

        def rw_body(rw, carry):
            pltpu.make_async_copy(
                itabT.at[pl.ds(0, 8), pl.ds(0, WIN)],
                win_v.at[0], semw).wait()

            w, r8, serve, cst = win_params(rw)
            buf = lax.rem(rw, DEPTH)
            o_lo = offs_s[w]
            o_hi = offs_s[w + 1]
            glo = o_lo // 16
            ghi = (o_hi + 15) // 16
            olo = jnp.full((16,), o_lo, jnp.int32)
            ohi = jnp.full((16,), o_hi, jnp.int32)
            bufv = jnp.full((16,), buf, jnp.int32)
            servev = jnp.full((16,), serve, jnp.int32)
            cstv = jnp.full((16,), cst, jnp.int32)
            r8v = jnp.full((16,), r8, jnp.int32)

            def grp2(g, carry2):
                pos = g * 16 + iota
                slots = ord_v[pl.ds(g * 16, 16)]
                iv = plsc.load_gather(ii_v, [slots])
                icol = iv - cstv
                valid = jnp.logical_and(
                    jnp.logical_and(pos >= olo, pos < ohi),
                    jnp.logical_and(iv >= servev, icol < WIN))
                icl = jnp.clip(icol, 0, WIN - 1)
                a = zero
                for jj in range(8):
                    jv = jnp.full((16,), jj, jnp.int32)
                    v = plsc.load_gather(win_v, [bufv, jv, icl])
                    u = plsc.load_gather(uall_v, [r8v + jv, slots])
                    a = a + jnp.where(
                        valid, jnp.maximum(v, 0.0) * u, 0.0)
                prev = plsc.load_gather(acc_v, [slots])
                plsc.store_scatter(acc_v, [slots], prev + a, mask=valid)
                return carry2

            lax.fori_loop(glo, ghi, grp2, 0)

            @pl.when(rw + DEPTH < 8 * NWIN)
            def _():
                start_win(rw + DEPTH)

            return carry

        lax.fori_loop(0, 8 * NWIN, rw_body, 0)

        @pl.when(wid == NW - 1)
        def _():
            def tailr(r, carry):
                pltpu.sync_copy(
                    itabT.at[pl.ds(r * 8, 8), pl.ds(TAILSTART, TAILW)],
                    tail_v)
                r8v = jnp.full((16,), r * 8, jnp.int32)

                def grp3(g, carry2):
                    e0 = g * 16
                    iv = ii_v[pl.ds(e0, 16)]
                    icol = iv - TAILSTART
                    valid = jnp.logical_and(
                        jnp.logical_and(icol >= 0, icol < TAILW),
                        e0 + iota < cround_v)
                    icl = jnp.clip(icol, 0, TAILW - 1)
                    a = zero
                    for jj in range(8):
                        jv = jnp.full((16,), jj, jnp.int32)
                        v = plsc.load_gather(tail_v, [jv, icl])
                        u = plsc.load_gather(uall_v, [r8v + jv, e0 + iota])
                        a = a + jnp.where(
                            valid, jnp.maximum(v, 0.0) * u, 0.0)
                    acc_v[pl.ds(e0, 16)] = acc_v[pl.ds(e0, 16)] + a
                    return carry2

                lax.fori_loop(0, ngrp, grp3, 0)
                return carry

            lax.fori_loop(0, 8, tailr, 0)

        def sch(ch, carry):
            pltpu.sync_copy(
                acc_v.at[pl.ds(ch * CHUNK, CHUNK)],
                shared_v.at[bo_v.at[ch]], add=True)
            return carry

        lax.fori_loop(0, nch, sch, 0)

    def cond(state):
        rnd, total = state
        return rnd * OCAP < total

    def roundbody(state):
        rnd, _ = state
        for k in range(DEPTH):
            start_win(k)
        total = scan_fill(rnd)
        process(total, rnd)
        return rnd + 1, total

    lax.while_loop(cond, roundbody, (0, 1))

    plsc.subcore_barrier()
    pltpu.sync_copy(
        shared_v.at[pl.ds(sid * (B // NS), B // NS)],
        part_out.at[core, pl.ds(sid * (B // NS), B // NS)])


def _body_merge(part_hbm, out_hbm, a_v, b_v, f_v, sem):
    wid = lax.axis_index("s") * NC + lax.axis_index("c")
    base = wid * BPW
    pltpu.sync_copy(part_hbm.at[0, pl.ds(base, BPW)], a_v)
    pltpu.sync_copy(part_hbm.at[1, pl.ds(base, BPW)], b_v)

    def grp(g, carry):
        f_v[pl.ds(g * 16, 16)] = (
            a_v[pl.ds(g * 16, 16)] + b_v[pl.ds(g * 16, 16)])
        return carry

    lax.fori_loop(0, BPW // 16, grp, 0)
    pltpu.sync_copy(f_v, out_hbm.at[pl.ds(base, BPW)])


@functools.partial(jax.jit, static_argnums=())
def _run(uidx, iidx, user_table, item_table):
    mesh = plsc.VectorSubcoreMesh(core_axis_name="c", subcore_axis_name="s")
    cp = pltpu.CompilerParams(
        needs_layout_passes=False, use_tc_tiling_on_sc=True,
        disable_bounds_checks=True)

    uidx2d = uidx.reshape(B // CHUNK, CHUNK)
    utab2 = user_table.reshape(NUM_USERS // 2, 2 * D)
    itabT = item_table.T

    ki = pl.kernel(
        _body_item, mesh=mesh,
        out_type=jax.ShapeDtypeStruct((NC, B), jnp.float32),
        scratch_types=[
            pltpu.VMEM((DEPTH, 8, WIN), jnp.float32),
            pltpu.VMEM((8, TAILW), jnp.float32),
            pltpu.VMEM((CHUNK, 2 * D), jnp.float32),
            pltpu.VMEM((D, OCAP), jnp.float32),
            pltpu.VMEM((B // NS,), jnp.float32),
            pltpu.VMEM((OCAP,), jnp.float32),
            pltpu.VMEM((OCAP,), jnp.int32),
            pltpu.VMEM((OCAP,), jnp.int32),
            pltpu.VMEM((OCAP // CHUNK, CHUNK), jnp.int32),
            pltpu.VMEM((OCAP // CHUNK, CHUNK), jnp.int32),
            pltpu.VMEM((OCAP,), jnp.int32),
            pltpu.VMEM((2, BLK), jnp.int32),
            pltpu.VMEM((2, BLK), jnp.int32),
            pltpu.SMEM((NWIN + 1,), jnp.int32),
            pltpu.VMEM_SHARED((B,), jnp.float32),
            pltpu.SemaphoreType.DMA,
            pltpu.SemaphoreType.DMA,
        ],
        compiler_params=cp)
    part = ki(uidx, iidx, itabT, utab2)

    km = pl.kernel(
        _body_merge, mesh=mesh,
        out_type=jax.ShapeDtypeStruct((B,), jnp.float32),
        scratch_types=[
            pltpu.VMEM((BPW,), jnp.float32),
            pltpu.VMEM((BPW,), jnp.float32),
            pltpu.VMEM((BPW,), jnp.float32),
            pltpu.SemaphoreType.DMA,
        ],
        compiler_params=cp)
    return km(part)


def kernel(user_indices, item_indices, user_table, item_table):
    return _run(user_indices.astype(jnp.int32),
                item_indices.astype(jnp.int32),
                user_table, item_table)
